# Initial kernel scaffold; baseline (speedup 1.0000x reference)
import jax
import jax.numpy as jnp
from jax.experimental import pallas as pl


def kernel(node_embeddings, w1_rel, w1_root, b1, w2_rel, w2_root, b2, fc1_w, fc1_b, fc2_w, fc2_b, edge_index_combined, edge_type_combined, bill_id, user1_id, user2_id):
    raise NotImplementedError("write your pallas kernel here")


# speedup vs baseline: 8.1771x; 8.1771x over previous
import functools

import jax
import jax.numpy as jnp
from jax import lax
from jax.experimental import pallas as pl
from jax.experimental.pallas import tpu as pltpu
from jax.experimental.pallas import tpu_sc as plsc

N = 10000
NP = 10240
R = 4
P = R * NP
D_IN = 128
D = 64
E = 320000
B = 4096
NC = 2
NS = 16
NW = NC * NS
EPW = 10240
EP = NW * EPW
CH = 128
NCHUNK = EPW // CH
PR = P // 128
PRP = 384

_mesh = functools.partial(
    plsc.VectorSubcoreMesh,
    core_axis_name="c", subcore_axis_name="s",
    num_cores=NC, num_subcores=NS,
)
_sc_params = pltpu.CompilerParams(use_tc_tiling_on_sc=False,
                                  needs_layout_passes=False)


def _counts_body(dst_hbm, et_hbm, out_hbm, dst_v, et_v, cv, ibuf, csh):
    cid = lax.axis_index("c")
    sid = lax.axis_index("s")
    wid = sid * NC + cid
    pltpu.sync_copy(dst_hbm.at[pl.ds(wid * NCHUNK, NCHUNK)], dst_v)
    pltpu.sync_copy(et_hbm.at[pl.ds(wid * NCHUNK, NCHUNK)], et_v)

    z16 = jnp.zeros((16,), jnp.int32)

    def zero_cv(g, _):
        r = g // 8
        c = g - r * 8
        cv[r, pl.ds(c * 16, 16)] = z16
        return 0
    lax.fori_loop(0, PRP * 8, zero_cv, 0)

    def mk_iota(g, _):
        j = g // 8
        l = g - j * 8
        ibuf[j, pl.ds(l * 16, 16)] = lax.iota(jnp.int32, 16) + g * 16
        return 0
    lax.fori_loop(0, (PRP // 128) * 8, mk_iota, 0)

    pltpu.sync_copy(cv.at[pl.ds(sid * (PRP // NS), PRP // NS)],
                    csh.at[pl.ds(sid * (PRP // NS), PRP // NS)])
    plsc.subcore_barrier()

    lane = lax.iota(jnp.int32, 16)
    one = jnp.ones((16,), jnp.int32)

    def count16(g, _):
        j = g // 8
        l = (g - j * 8) * 16
        sl = pl.ds(l, 16)
        kvec = et_v[j, sl] * NP + dst_v[j, sl]
        for i in range(16):
            k = kvec[i]
            r = k // 128
            cg = (k // 16) - r * 8
            c = k - (k // 16) * 16
            oh = jnp.where((lane - c) == 0, one, z16)
            csl = pl.ds(cg * 16, 16)
            cv[r, csl] = cv[r, csl] + oh
        return 0
    lax.fori_loop(0, EPW // 16, count16, 0)

    def add_chunk(j, _):
        pltpu.sync_copy(cv.at[pl.ds(j * 128, 128)], csh.at[ibuf.at[j]],
                        add=True)
        return 0
    lax.fori_loop(0, PRP // 128, add_chunk, 0)
    plsc.subcore_barrier()

    base = sid * (PR // NS)
    pltpu.sync_copy(csh.at[pl.ds(base, PR // NS)], cv.at[pl.ds(0, PR // NS)])
    pltpu.sync_copy(cv.at[pl.ds(0, PR // NS)],
                    out_hbm.at[pl.ds(cid * PR + base, PR // NS)])


def _counts_call(dst3, et3):
    return pl.kernel(
        _counts_body,
        out_type=jax.ShapeDtypeStruct((NC * PR, 128), jnp.int32),
        mesh=_mesh(),
        compiler_params=_sc_params,
        scratch_types=[
            pltpu.VMEM((NCHUNK, CH), jnp.int32),
            pltpu.VMEM((NCHUNK, CH), jnp.int32),
            pltpu.VMEM((PRP, 128), jnp.int32),
            pltpu.VMEM((PRP // 128, 128), jnp.int32),
            pltpu.VMEM_SHARED((PRP, 128), jnp.int32),
        ],
    )(dst3, et3)


def _edge_body(t_hbm, src_hbm, dst_hbm, et_hbm, invc_hbm, acc_hbm,
               gk_v, dst_v, et_v, s_v, invc_v, rows_v, acc_sh, sem):
    cid = lax.axis_index("c")
    sid = lax.axis_index("s")
    wid = sid * NC + cid
    pltpu.sync_copy(src_hbm.at[pl.ds(wid * NCHUNK, NCHUNK)], gk_v)
    pltpu.sync_copy(dst_hbm.at[pl.ds(wid * NCHUNK, NCHUNK)], dst_v)
    pltpu.sync_copy(et_hbm.at[pl.ds(wid * NCHUNK, NCHUNK)], et_v)
    pltpu.sync_copy(invc_hbm, invc_v)

    zv = jnp.zeros((16,), jnp.float32)

    def zero_rows(g, _):
        r = g // 4
        c = g - r * 4
        rows_v[r, pl.ds(c * 16, 16)] = zv
        return 0
    lax.fori_loop(0, CH * 4, zero_rows, 0)

    def zero_acc(t, _):
        pltpu.sync_copy(rows_v, acc_sh.at[pl.ds(sid * (NP // NS) + t * CH, CH)])
        return 0
    lax.fori_loop(0, NP // NS // CH, zero_acc, 0)

    def keys_scales(g, _):
        j = g // 8
        l = (g - j * 8) * 16
        sl = pl.ds(l, 16)
        etv = et_v[j, sl] * NP
        sk = etv + dst_v[j, sl]
        gk_v[j, sl] = gk_v[j, sl] + etv
        s_v[j, sl] = plsc.load_gather(invc_v, [sk])
        return 0
    lax.fori_loop(0, EPW // 16, keys_scales, 0)
    plsc.subcore_barrier()

    def chunk(j, _):
        pltpu.async_copy(t_hbm.at[gk_v.at[j]], rows_v, sem).wait()

        def scale16(eb, _):
            svec = s_v[j, pl.ds(eb * 16, 16)]
            for i in range(16):
                sval = svec[i]
                e = eb * 16 + i
                for c in range(4):
                    sl = pl.ds(c * 16, 16)
                    rows_v[e, sl] = rows_v[e, sl] * sval
            return 0
        lax.fori_loop(0, CH // 16, scale16, 0)
        pltpu.sync_copy(rows_v, acc_sh.at[dst_v.at[j]], add=True)
        return 0
    lax.fori_loop(0, NCHUNK, chunk, 0)
    plsc.subcore_barrier()

    def export(t, _):
        base = sid * (NP // NS) + t * CH
        pltpu.sync_copy(acc_sh.at[pl.ds(base, CH)], rows_v)
        pltpu.sync_copy(rows_v, acc_hbm.at[pl.ds(cid * NP + base, CH)])
        return 0
    lax.fori_loop(0, NP // NS // CH, export, 0)


def _edge_call(t_tab, src3, dst3, et3, invc):
    return pl.kernel(
        _edge_body,
        out_type=jax.ShapeDtypeStruct((NC * NP, D), jnp.float32),
        mesh=_mesh(),
        compiler_params=_sc_params,
        scratch_types=[
            pltpu.VMEM((NCHUNK, CH), jnp.int32),
            pltpu.VMEM((NCHUNK, CH), jnp.int32),
            pltpu.VMEM((NCHUNK, CH), jnp.int32),
            pltpu.VMEM((NCHUNK, CH), jnp.float32),
            pltpu.VMEM((P,), jnp.float32),
            pltpu.VMEM((CH, D), jnp.float32),
            pltpu.VMEM_SHARED((NP, D), jnp.float32),
            pltpu.SemaphoreType.DMA,
        ],
    )(t_tab, src3, dst3, et3, invc)


def _head_gather_body(idx_hbm, root_hbm, a0_hbm, a1_hbm, out_hbm,
                      idx_v, r0, r1, r2, sem):
    cid = lax.axis_index("c")
    sid = lax.axis_index("s")
    wid = sid * NC + cid
    pltpu.sync_copy(idx_hbm.at[pl.ds(wid * 3, 3)], idx_v)

    def chunk(j, _):
        pltpu.async_copy(root_hbm.at[idx_v.at[j]], r0, sem).wait()
        pltpu.async_copy(a0_hbm.at[idx_v.at[j]], r1, sem).wait()
        pltpu.async_copy(a1_hbm.at[idx_v.at[j]], r2, sem).wait()

        def addr(r, _):
            for c in range(4):
                sl = pl.ds(c * 16, 16)
                r0[r, sl] = r0[r, sl] + r1[r, sl] + r2[r, sl]
            return 0
        lax.fori_loop(0, CH, addr, 0)
        pltpu.sync_copy(r0, out_hbm.at[pl.ds(wid * 384 + j * CH, CH)])
        return 0
    lax.fori_loop(0, 3, chunk, 0)


def _head_gather_call(idx3, root2, a0, a1):
    return pl.kernel(
        _head_gather_body,
        out_type=jax.ShapeDtypeStruct((3 * B, D), jnp.float32),
        mesh=_mesh(),
        compiler_params=_sc_params,
        scratch_types=[
            pltpu.VMEM((3, CH), jnp.int32),
            pltpu.VMEM((CH, D), jnp.float32),
            pltpu.VMEM((CH, D), jnp.float32),
            pltpu.VMEM((CH, D), jnp.float32),
            pltpu.SemaphoreType.DMA,
        ],
    )(idx3, root2, a0, a1)


def _prep1_body(x_ref, w_ref, wr_ref, b_ref, cnt_ref, t_ref, root_ref,
                invc_ref):
    x = x_ref[...]
    t_ref[0:N, :] = jnp.dot(x, w_ref[0], preferred_element_type=jnp.float32)
    t_ref[N:NP, :] = jnp.zeros((NP - N, D), jnp.float32)
    root_ref[...] = (
        jnp.dot(x, wr_ref[...], preferred_element_type=jnp.float32)
        + b_ref[...]
    )
    cnt = cnt_ref[...]
    c = (cnt[0] + cnt[1]).astype(jnp.float32)
    rows = P // 128
    k = (lax.broadcasted_iota(jnp.int32, (rows, 128), 0) * 128
         + lax.broadcasted_iota(jnp.int32, (rows, 128), 1))
    npart = k % NP
    invc_ref[...] = jnp.where(npart < N, 1.0 / jnp.maximum(c, 1.0), 0.0)


def _prep1_call(x, w1_rel, w1_root, b1r, cnt2):
    rows = P // 128
    return pl.pallas_call(
        _prep1_body,
        grid=(R,),
        in_specs=[
            pl.BlockSpec((N, D_IN), lambda r: (0, 0)),
            pl.BlockSpec((1, D_IN, D), lambda r: (r, 0, 0)),
            pl.BlockSpec((D_IN, D), lambda r: (0, 0)),
            pl.BlockSpec((1, D), lambda r: (0, 0)),
            pl.BlockSpec((2, rows, 128), lambda r: (0, 0, 0)),
        ],
        out_specs=[
            pl.BlockSpec((NP, D), lambda r: (r, 0)),
            pl.BlockSpec((N, D), lambda r: (0, 0)),
            pl.BlockSpec((rows, 128), lambda r: (0, 0)),
        ],
        out_shape=[
            jax.ShapeDtypeStruct((P, D), jnp.float32),
            jax.ShapeDtypeStruct((N, D), jnp.float32),
            jax.ShapeDtypeStruct((rows, 128), jnp.float32),
        ],
    )(x, w1_rel, w1_root, b1r, cnt2)


def _mid_body(root1_ref, acc_ref, w_ref, wr_ref, b_ref, t_ref, root2_ref):
    a = acc_ref[...]
    h = root1_ref[...] + a[0:N, :] + a[NP:NP + N, :]
    h = jnp.maximum(h, 0.0)
    t_ref[0:N, :] = jnp.dot(h, w_ref[0], preferred_element_type=jnp.float32)
    t_ref[N:NP, :] = jnp.zeros((NP - N, D), jnp.float32)
    root2_ref[...] = (
        jnp.dot(h, wr_ref[...], preferred_element_type=jnp.float32)
        + b_ref[...]
    )


def _mid_call(root1, acc1, w2_rel, w2_root, b2r):
    return pl.pallas_call(
        _mid_body,
        grid=(R,),
        in_specs=[
            pl.BlockSpec((N, D), lambda r: (0, 0)),
            pl.BlockSpec((NC * NP, D), lambda r: (0, 0)),
            pl.BlockSpec((1, D, D), lambda r: (r, 0, 0)),
            pl.BlockSpec((D, D), lambda r: (0, 0)),
            pl.BlockSpec((1, D), lambda r: (0, 0)),
        ],
        out_specs=[
            pl.BlockSpec((NP, D), lambda r: (r, 0)),
            pl.BlockSpec((N, D), lambda r: (0, 0)),
        ],
        out_shape=[
            jax.ShapeDtypeStruct((P, D), jnp.float32),
            jax.ShapeDtypeStruct((N, D), jnp.float32),
        ],
    )(root1, acc1, w2_rel, w2_root, b2r)


def _head_body(rows_ref, w1_ref, b1_ref, w2t_ref, b2_ref, out_ref):
    xr = rows_ref[...]
    bill = xr[0:B]
    u1 = xr[B:2 * B]
    u2 = xr[2 * B:3 * B]
    w2b = jnp.broadcast_to(w2t_ref[...], (D, 128))
    b2v = b2_ref[...]

    def mlp(a, b):
        xcat = jnp.concatenate([a, b], axis=1)
        h1 = lax.dot_general(xcat, w1_ref[...], (((1,), (1,)), ((), ())),
                             preferred_element_type=jnp.float32)
        h1 = jnp.maximum(h1 + b1_ref[...], 0.0)
        return jnp.dot(h1, w2b, preferred_element_type=jnp.float32) + b2v[0, 0]

    p = mlp(bill, u1)
    q = mlp(bill, u2)

    def softplus(v):
        return jnp.maximum(v, 0.0) + jnp.log(1.0 + jnp.exp(-jnp.abs(v)))

    v = (softplus(p) - p) + softplus(q)
    s = jnp.sum(v, axis=0, keepdims=True)
    out_ref[...] = s[0:1, 0:1] / (2.0 * B)


def _head_call(rows3, fc1_w, fc1_br, fc2_wt, fc2_br):
    return pl.pallas_call(
        _head_body,
        out_shape=jax.ShapeDtypeStruct((1, 1), jnp.float32),
    )(rows3, fc1_w, fc1_br, fc2_wt, fc2_br)


def kernel(node_embeddings, w1_rel, w1_root, b1, w2_rel, w2_root, b2,
           fc1_w, fc1_b, fc2_w, fc2_b,
           edge_index_combined, edge_type_combined,
           bill_id, user1_id, user2_id):
    src = edge_index_combined[0].astype(jnp.int32)
    dst = edge_index_combined[1].astype(jnp.int32)
    et = edge_type_combined.astype(jnp.int32)
    pad = EP - E
    src_p = jnp.concatenate([src, jnp.zeros((pad,), jnp.int32)])
    dst_p = jnp.concatenate([dst, jnp.full((pad,), N, jnp.int32)])
    et_p = jnp.concatenate([et, jnp.zeros((pad,), jnp.int32)])
    src3 = src_p.reshape(NW * NCHUNK, CH)
    dst3 = dst_p.reshape(NW * NCHUNK, CH)
    et3 = et_p.reshape(NW * NCHUNK, CH)

    cntp = _counts_call(dst3, et3)
    cnt2 = cntp.reshape(2, P // 128, 128)

    t1, root1, invc2 = _prep1_call(
        node_embeddings, w1_rel, w1_root, b1.reshape(1, D), cnt2)
    invc = invc2.reshape(P)

    acc1 = _edge_call(t1, src3, dst3, et3, invc)
    t2, root2 = _mid_call(root1, acc1, w2_rel, w2_root, b2.reshape(1, D))
    acc2 = _edge_call(t2, src3, dst3, et3, invc)

    idx3 = jnp.concatenate([
        bill_id.astype(jnp.int32),
        user1_id.astype(jnp.int32),
        user2_id.astype(jnp.int32),
    ]).reshape(NW * 3, CH)
    rows3 = _head_gather_call(idx3, root2, acc2[0:NP], acc2[NP:])

    loss = _head_call(rows3, fc1_w, fc1_b.reshape(1, D),
                      fc2_w.reshape(D, 1), fc2_b.reshape(1, 1))
    return jnp.reshape(loss, ())


# speedup vs baseline: 11.5414x; 1.4114x over previous
import functools

import jax
import jax.numpy as jnp
from jax import lax
from jax.experimental import pallas as pl
from jax.experimental.pallas import tpu as pltpu
from jax.experimental.pallas import tpu_sc as plsc

N = 10000
NP = 10240
R = 4
P = R * NP
D_IN = 128
D = 64
E = 320000
B = 4096
NC = 2
NS = 16
NW = NC * NS
EPW = 10240
EP = NW * EPW
CH = 128
NCHUNK = EPW // CH
PR = P // 128
PRP = 384

_mesh = functools.partial(
    plsc.VectorSubcoreMesh,
    core_axis_name="c", subcore_axis_name="s",
    num_cores=NC, num_subcores=NS,
)
_sc_params = pltpu.CompilerParams(use_tc_tiling_on_sc=False,
                                  needs_layout_passes=False)


def _counts_body(dst_hbm, et_hbm, out_hbm, dst_v, et_v, cv, ibuf, csh):
    cid = lax.axis_index("c")
    sid = lax.axis_index("s")
    wid = sid * NC + cid
    pltpu.sync_copy(dst_hbm.at[pl.ds(wid * NCHUNK, NCHUNK)], dst_v)
    pltpu.sync_copy(et_hbm.at[pl.ds(wid * NCHUNK, NCHUNK)], et_v)

    z16 = jnp.zeros((16,), jnp.int32)

    def zero_cv(g, _):
        r = g // 8
        c = g - r * 8
        cv[r, pl.ds(c * 16, 16)] = z16
        return 0
    lax.fori_loop(0, PRP * 8, zero_cv, 0)

    def mk_iota(g, _):
        j = g // 8
        l = g - j * 8
        ibuf[j, pl.ds(l * 16, 16)] = lax.iota(jnp.int32, 16) + g * 16
        return 0
    lax.fori_loop(0, (PRP // 128) * 8, mk_iota, 0)

    pltpu.sync_copy(cv.at[pl.ds(sid * (PRP // NS), PRP // NS)],
                    csh.at[pl.ds(sid * (PRP // NS), PRP // NS)])
    plsc.subcore_barrier()

    lane = lax.iota(jnp.int32, 16)
    one = jnp.ones((16,), jnp.int32)

    def count16(g, _):
        j = g // 8
        l = (g - j * 8) * 16
        sl = pl.ds(l, 16)
        kvec = et_v[j, sl] * NP + dst_v[j, sl]
        for i in range(16):
            k = kvec[i]
            r = k // 128
            cg = (k // 16) - r * 8
            c = k - (k // 16) * 16
            oh = jnp.where((lane - c) == 0, one, z16)
            csl = pl.ds(cg * 16, 16)
            cv[r, csl] = cv[r, csl] + oh
        return 0
    lax.fori_loop(0, EPW // 16, count16, 0)

    def add_chunk(j, _):
        pltpu.sync_copy(cv.at[pl.ds(j * 128, 128)], csh.at[ibuf.at[j]],
                        add=True)
        return 0
    lax.fori_loop(0, PRP // 128, add_chunk, 0)
    plsc.subcore_barrier()

    base = sid * (PR // NS)
    pltpu.sync_copy(csh.at[pl.ds(base, PR // NS)], cv.at[pl.ds(0, PR // NS)])
    pltpu.sync_copy(cv.at[pl.ds(0, PR // NS)],
                    out_hbm.at[pl.ds(cid * PR + base, PR // NS)])


def _counts_call(dst3, et3):
    return pl.kernel(
        _counts_body,
        out_type=jax.ShapeDtypeStruct((NC * PR, 128), jnp.int32),
        mesh=_mesh(),
        compiler_params=_sc_params,
        scratch_types=[
            pltpu.VMEM((NCHUNK, CH), jnp.int32),
            pltpu.VMEM((NCHUNK, CH), jnp.int32),
            pltpu.VMEM((PRP, 128), jnp.int32),
            pltpu.VMEM((PRP // 128, 128), jnp.int32),
            pltpu.VMEM_SHARED((PRP, 128), jnp.int32),
        ],
    )(dst3, et3)


def _scales_body(src_hbm, dst_hbm, et_hbm, invc_hbm, gk_hbm, s_hbm,
                 gk_v, dst_v, et_v, s_v, invc_v):
    cid = lax.axis_index("c")
    sid = lax.axis_index("s")
    wid = sid * NC + cid
    pltpu.sync_copy(src_hbm.at[pl.ds(wid * NCHUNK, NCHUNK)], gk_v)
    pltpu.sync_copy(dst_hbm.at[pl.ds(wid * NCHUNK, NCHUNK)], dst_v)
    pltpu.sync_copy(et_hbm.at[pl.ds(wid * NCHUNK, NCHUNK)], et_v)
    pltpu.sync_copy(invc_hbm, invc_v)

    def keys_scales(g, _):
        j = g // 8
        l = (g - j * 8) * 16
        sl = pl.ds(l, 16)
        etv = et_v[j, sl] * NP
        sk = etv + dst_v[j, sl]
        gk_v[j, sl] = gk_v[j, sl] + etv
        s_v[j, sl] = plsc.load_gather(invc_v, [sk])
        return 0
    lax.fori_loop(0, EPW // 16, keys_scales, 0)
    pltpu.sync_copy(gk_v, gk_hbm.at[pl.ds(wid * NCHUNK, NCHUNK)])
    pltpu.sync_copy(s_v, s_hbm.at[pl.ds(wid * NCHUNK, NCHUNK)])


def _scales_call(src3, dst3, et3, invc):
    return pl.kernel(
        _scales_body,
        out_type=(
            jax.ShapeDtypeStruct((NW * NCHUNK, CH), jnp.int32),
            jax.ShapeDtypeStruct((NW * NCHUNK, CH), jnp.float32),
        ),
        mesh=_mesh(),
        compiler_params=_sc_params,
        scratch_types=[
            pltpu.VMEM((NCHUNK, CH), jnp.int32),
            pltpu.VMEM((NCHUNK, CH), jnp.int32),
            pltpu.VMEM((NCHUNK, CH), jnp.int32),
            pltpu.VMEM((NCHUNK, CH), jnp.float32),
            pltpu.VMEM((P,), jnp.float32),
        ],
    )(src3, dst3, et3, invc)


def _edge_body(t_hbm, gk_hbm, dst_hbm, s_hbm, acc_hbm,
               gk_v, dst_v, s_v, bufs, gsems, ssems, acc_sh):
    cid = lax.axis_index("c")
    sid = lax.axis_index("s")
    wid = sid * NC + cid
    pltpu.sync_copy(gk_hbm.at[pl.ds(wid * NCHUNK, NCHUNK)], gk_v)
    pltpu.sync_copy(dst_hbm.at[pl.ds(wid * NCHUNK, NCHUNK)], dst_v)
    pltpu.sync_copy(s_hbm.at[pl.ds(wid * NCHUNK, NCHUNK)], s_v)

    zv = jnp.zeros((16,), jnp.float32)
    rows0 = bufs[0]

    def zero_rows(g, _):
        r = g // 4
        c = g - r * 4
        rows0[r, pl.ds(c * 16, 16)] = zv
        return 0
    lax.fori_loop(0, CH * 4, zero_rows, 0)

    def zero_acc(t, _):
        pltpu.sync_copy(rows0, acc_sh.at[pl.ds(sid * (NP // NS) + t * CH, CH)])
        return 0
    lax.fori_loop(0, NP // NS // CH, zero_acc, 0)
    plsc.subcore_barrier()

    def scale_chunk(rv, j):
        def scale16(eb, _):
            svec = s_v[j, pl.ds(eb * 16, 16)]
            for i in range(16):
                sval = svec[i]
                e = eb * 16 + i
                for c in range(4):
                    sl = pl.ds(c * 16, 16)
                    rv[e, sl] = rv[e, sl] * sval
            return 0
        lax.fori_loop(0, CH // 16, scale16, 0)

    pltpu.async_copy(t_hbm.at[gk_v.at[0]], bufs[0], gsems[0])
    pltpu.async_copy(t_hbm.at[gk_v.at[1]], bufs[1], gsems[1])

    def chunk4(t, _):
        for p in range(4):
            j = t * 4 + p
            b = p
            nb = (p + 2) % 4
            nj = j + 2

            @pl.when(nj < NCHUNK)
            def _():
                @pl.when(j >= 2)
                def _():
                    pltpu.make_async_copy(bufs[nb], acc_sh.at[dst_v.at[j - 2]],
                                          ssems[nb]).wait()
                pltpu.async_copy(t_hbm.at[gk_v.at[nj]], bufs[nb], gsems[nb])
            pltpu.make_async_copy(t_hbm.at[gk_v.at[j]], bufs[b],
                                  gsems[b]).wait()
            scale_chunk(bufs[b], j)
            pltpu.async_copy(bufs[b], acc_sh.at[dst_v.at[j]], ssems[b],
                             add=True)
        return 0
    lax.fori_loop(0, NCHUNK // 4, chunk4, 0)
    pltpu.make_async_copy(bufs[(NCHUNK - 2) % 4],
                          acc_sh.at[dst_v.at[NCHUNK - 2]],
                          ssems[(NCHUNK - 2) % 4]).wait()
    pltpu.make_async_copy(bufs[(NCHUNK - 1) % 4],
                          acc_sh.at[dst_v.at[NCHUNK - 1]],
                          ssems[(NCHUNK - 1) % 4]).wait()
    plsc.subcore_barrier()

    def export(t, _):
        base = sid * (NP // NS) + t * CH
        pltpu.sync_copy(acc_sh.at[pl.ds(base, CH)], rows0)
        pltpu.sync_copy(rows0, acc_hbm.at[pl.ds(cid * NP + base, CH)])
        return 0
    lax.fori_loop(0, NP // NS // CH, export, 0)


def _edge_call(t_tab, gk3, dst3, s3):
    return pl.kernel(
        _edge_body,
        out_type=jax.ShapeDtypeStruct((NC * NP, D), jnp.float32),
        mesh=_mesh(),
        compiler_params=_sc_params,
        scratch_types=[
            pltpu.VMEM((NCHUNK, CH), jnp.int32),
            pltpu.VMEM((NCHUNK, CH), jnp.int32),
            pltpu.VMEM((NCHUNK, CH), jnp.float32),
            [pltpu.VMEM((CH, D), jnp.float32) for _ in range(4)],
            [pltpu.SemaphoreType.DMA for _ in range(4)],
            [pltpu.SemaphoreType.DMA for _ in range(4)],
            pltpu.VMEM_SHARED((NP, D), jnp.float32),
        ],
    )(t_tab, gk3, dst3, s3)


def _head_gather_body(idx_hbm, root_hbm, a0_hbm, a1_hbm, out_hbm,
                      idx_v, r0, r1, r2, sem):
    cid = lax.axis_index("c")
    sid = lax.axis_index("s")
    wid = sid * NC + cid
    pltpu.sync_copy(idx_hbm.at[pl.ds(wid * 3, 3)], idx_v)

    def chunk(j, _):
        pltpu.async_copy(root_hbm.at[idx_v.at[j]], r0, sem).wait()
        pltpu.async_copy(a0_hbm.at[idx_v.at[j]], r1, sem).wait()
        pltpu.async_copy(a1_hbm.at[idx_v.at[j]], r2, sem).wait()

        def addr(r, _):
            for c in range(4):
                sl = pl.ds(c * 16, 16)
                r0[r, sl] = r0[r, sl] + r1[r, sl] + r2[r, sl]
            return 0
        lax.fori_loop(0, CH, addr, 0)
        pltpu.sync_copy(r0, out_hbm.at[pl.ds(wid * 384 + j * CH, CH)])
        return 0
    lax.fori_loop(0, 3, chunk, 0)


def _head_gather_call(idx3, root2, a0, a1):
    return pl.kernel(
        _head_gather_body,
        out_type=jax.ShapeDtypeStruct((3 * B, D), jnp.float32),
        mesh=_mesh(),
        compiler_params=_sc_params,
        scratch_types=[
            pltpu.VMEM((3, CH), jnp.int32),
            pltpu.VMEM((CH, D), jnp.float32),
            pltpu.VMEM((CH, D), jnp.float32),
            pltpu.VMEM((CH, D), jnp.float32),
            pltpu.SemaphoreType.DMA,
        ],
    )(idx3, root2, a0, a1)


def _prep1_body(x_ref, w_ref, wr_ref, b_ref, cnt_ref, t_ref, root_ref,
                invc_ref):
    x = x_ref[...]
    t_ref[0:N, :] = jnp.dot(x, w_ref[0], preferred_element_type=jnp.float32)
    t_ref[N:NP, :] = jnp.zeros((NP - N, D), jnp.float32)
    root_ref[...] = (
        jnp.dot(x, wr_ref[...], preferred_element_type=jnp.float32)
        + b_ref[...]
    )
    cnt = cnt_ref[...]
    c = (cnt[0] + cnt[1]).astype(jnp.float32)
    rows = P // 128
    k = (lax.broadcasted_iota(jnp.int32, (rows, 128), 0) * 128
         + lax.broadcasted_iota(jnp.int32, (rows, 128), 1))
    npart = k % NP
    invc_ref[...] = jnp.where(npart < N, 1.0 / jnp.maximum(c, 1.0), 0.0)


def _prep1_call(x, w1_rel, w1_root, b1r, cnt2):
    rows = P // 128
    return pl.pallas_call(
        _prep1_body,
        grid=(R,),
        in_specs=[
            pl.BlockSpec((N, D_IN), lambda r: (0, 0)),
            pl.BlockSpec((1, D_IN, D), lambda r: (r, 0, 0)),
            pl.BlockSpec((D_IN, D), lambda r: (0, 0)),
            pl.BlockSpec((1, D), lambda r: (0, 0)),
            pl.BlockSpec((2, rows, 128), lambda r: (0, 0, 0)),
        ],
        out_specs=[
            pl.BlockSpec((NP, D), lambda r: (r, 0)),
            pl.BlockSpec((N, D), lambda r: (0, 0)),
            pl.BlockSpec((rows, 128), lambda r: (0, 0)),
        ],
        out_shape=[
            jax.ShapeDtypeStruct((P, D), jnp.float32),
            jax.ShapeDtypeStruct((N, D), jnp.float32),
            jax.ShapeDtypeStruct((rows, 128), jnp.float32),
        ],
    )(x, w1_rel, w1_root, b1r, cnt2)


def _mid_body(root1_ref, acc_ref, w_ref, wr_ref, b_ref, t_ref, root2_ref):
    a = acc_ref[...]
    h = root1_ref[...] + a[0:N, :] + a[NP:NP + N, :]
    h = jnp.maximum(h, 0.0)
    t_ref[0:N, :] = jnp.dot(h, w_ref[0], preferred_element_type=jnp.float32)
    t_ref[N:NP, :] = jnp.zeros((NP - N, D), jnp.float32)
    root2_ref[...] = (
        jnp.dot(h, wr_ref[...], preferred_element_type=jnp.float32)
        + b_ref[...]
    )


def _mid_call(root1, acc1, w2_rel, w2_root, b2r):
    return pl.pallas_call(
        _mid_body,
        grid=(R,),
        in_specs=[
            pl.BlockSpec((N, D), lambda r: (0, 0)),
            pl.BlockSpec((NC * NP, D), lambda r: (0, 0)),
            pl.BlockSpec((1, D, D), lambda r: (r, 0, 0)),
            pl.BlockSpec((D, D), lambda r: (0, 0)),
            pl.BlockSpec((1, D), lambda r: (0, 0)),
        ],
        out_specs=[
            pl.BlockSpec((NP, D), lambda r: (r, 0)),
            pl.BlockSpec((N, D), lambda r: (0, 0)),
        ],
        out_shape=[
            jax.ShapeDtypeStruct((P, D), jnp.float32),
            jax.ShapeDtypeStruct((N, D), jnp.float32),
        ],
    )(root1, acc1, w2_rel, w2_root, b2r)


def _head_body(rows_ref, w1_ref, b1_ref, w2t_ref, b2_ref, out_ref):
    xr = rows_ref[...]
    bill = xr[0:B]
    u1 = xr[B:2 * B]
    u2 = xr[2 * B:3 * B]
    w2b = jnp.broadcast_to(w2t_ref[...], (D, 128))
    b2v = b2_ref[...]

    def mlp(a, b):
        xcat = jnp.concatenate([a, b], axis=1)
        h1 = lax.dot_general(xcat, w1_ref[...], (((1,), (1,)), ((), ())),
                             preferred_element_type=jnp.float32)
        h1 = jnp.maximum(h1 + b1_ref[...], 0.0)
        return jnp.dot(h1, w2b, preferred_element_type=jnp.float32) + b2v[0, 0]

    p = mlp(bill, u1)
    q = mlp(bill, u2)

    def softplus(v):
        return jnp.maximum(v, 0.0) + jnp.log(1.0 + jnp.exp(-jnp.abs(v)))

    v = (softplus(p) - p) + softplus(q)
    s = jnp.sum(v, axis=0, keepdims=True)
    out_ref[...] = s[0:1, 0:1] / (2.0 * B)


def _head_call(rows3, fc1_w, fc1_br, fc2_wt, fc2_br):
    return pl.pallas_call(
        _head_body,
        out_shape=jax.ShapeDtypeStruct((1, 1), jnp.float32),
    )(rows3, fc1_w, fc1_br, fc2_wt, fc2_br)


def kernel(node_embeddings, w1_rel, w1_root, b1, w2_rel, w2_root, b2,
           fc1_w, fc1_b, fc2_w, fc2_b,
           edge_index_combined, edge_type_combined,
           bill_id, user1_id, user2_id):
    src = edge_index_combined[0].astype(jnp.int32)
    dst = edge_index_combined[1].astype(jnp.int32)
    et = edge_type_combined.astype(jnp.int32)
    pad = EP - E
    src_p = jnp.concatenate([src, jnp.zeros((pad,), jnp.int32)])
    dst_p = jnp.concatenate([dst, jnp.full((pad,), N, jnp.int32)])
    et_p = jnp.concatenate([et, jnp.zeros((pad,), jnp.int32)])
    src3 = src_p.reshape(NW * NCHUNK, CH)
    dst3 = dst_p.reshape(NW * NCHUNK, CH)
    et3 = et_p.reshape(NW * NCHUNK, CH)

    cntp = _counts_call(dst3, et3)
    cnt2 = cntp.reshape(2, P // 128, 128)

    t1, root1, invc2 = _prep1_call(
        node_embeddings, w1_rel, w1_root, b1.reshape(1, D), cnt2)
    invc = invc2.reshape(P)

    gk3, s3 = _scales_call(src3, dst3, et3, invc)
    acc1 = _edge_call(t1, gk3, dst3, s3)
    t2, root2 = _mid_call(root1, acc1, w2_rel, w2_root, b2.reshape(1, D))
    acc2 = _edge_call(t2, gk3, dst3, s3)

    idx3 = jnp.concatenate([
        bill_id.astype(jnp.int32),
        user1_id.astype(jnp.int32),
        user2_id.astype(jnp.int32),
    ]).reshape(NW * 3, CH)
    rows3 = _head_gather_call(idx3, root2, acc2[0:NP], acc2[NP:])

    loss = _head_call(rows3, fc1_w, fc1_b.reshape(1, D),
                      fc2_w.reshape(D, 1), fc2_b.reshape(1, 1))
    return jnp.reshape(loss, ())


# speedup vs baseline: 11.5462x; 1.0004x over previous
import functools

import jax
import jax.numpy as jnp
from jax import lax
from jax.experimental import pallas as pl
from jax.experimental.pallas import tpu as pltpu
from jax.experimental.pallas import tpu_sc as plsc

N = 10000
NP = 10240
R = 4
P = R * NP
D_IN = 128
D = 64
E = 320000
B = 4096
NC = 2
NS = 16
NW = NC * NS
EPW = 10240
EP = NW * EPW
CH = 128
NCHUNK = EPW // CH
CH2 = 256
NCH2 = EPW // CH2
PR = P // 128
PRP = 384

_mesh = functools.partial(
    plsc.VectorSubcoreMesh,
    core_axis_name="c", subcore_axis_name="s",
    num_cores=NC, num_subcores=NS,
)
_sc_params = pltpu.CompilerParams(use_tc_tiling_on_sc=False,
                                  needs_layout_passes=False)


def _counts_body(dst_hbm, et_hbm, out_hbm, dst_v, et_v, cv, ibuf, csh):
    cid = lax.axis_index("c")
    sid = lax.axis_index("s")
    wid = sid * NC + cid
    pltpu.sync_copy(dst_hbm.at[pl.ds(wid * NCH2, NCH2)], dst_v)
    pltpu.sync_copy(et_hbm.at[pl.ds(wid * NCH2, NCH2)], et_v)

    z16 = jnp.zeros((16,), jnp.int32)

    def zero_cv(g, _):
        r = g // 8
        c = g - r * 8
        cv[r, pl.ds(c * 16, 16)] = z16
        return 0
    lax.fori_loop(0, PRP * 8, zero_cv, 0)

    def mk_iota(g, _):
        j = g // 8
        l = g - j * 8
        ibuf[j, pl.ds(l * 16, 16)] = lax.iota(jnp.int32, 16) + g * 16
        return 0
    lax.fori_loop(0, (PRP // 128) * 8, mk_iota, 0)

    pltpu.sync_copy(cv.at[pl.ds(sid * (PRP // NS), PRP // NS)],
                    csh.at[pl.ds(sid * (PRP // NS), PRP // NS)])
    plsc.subcore_barrier()

    lane = lax.iota(jnp.int32, 16)
    one = jnp.ones((16,), jnp.int32)

    def count16(g, _):
        j = g // 16
        l = (g - j * 16) * 16
        sl = pl.ds(l, 16)
        kvec = et_v[j, sl] * NP + dst_v[j, sl]
        for i in range(16):
            k = kvec[i]
            r = k // 128
            cg = (k // 16) - r * 8
            c = k - (k // 16) * 16
            oh = jnp.where((lane - c) == 0, one, z16)
            csl = pl.ds(cg * 16, 16)
            cv[r, csl] = cv[r, csl] + oh
        return 0
    lax.fori_loop(0, EPW // 16, count16, 0)

    def add_chunk(j, _):
        pltpu.sync_copy(cv.at[pl.ds(j * 128, 128)], csh.at[ibuf.at[j]],
                        add=True)
        return 0
    lax.fori_loop(0, PRP // 128, add_chunk, 0)
    plsc.subcore_barrier()

    base = sid * (PR // NS)
    pltpu.sync_copy(csh.at[pl.ds(base, PR // NS)], cv.at[pl.ds(0, PR // NS)])
    pltpu.sync_copy(cv.at[pl.ds(0, PR // NS)],
                    out_hbm.at[pl.ds(cid * PR + base, PR // NS)])


def _counts_call(dst3, et3):
    return pl.kernel(
        _counts_body,
        out_type=jax.ShapeDtypeStruct((NC * PR, 128), jnp.int32),
        mesh=_mesh(),
        compiler_params=_sc_params,
        scratch_types=[
            pltpu.VMEM((NCH2, CH2), jnp.int32),
            pltpu.VMEM((NCH2, CH2), jnp.int32),
            pltpu.VMEM((PRP, 128), jnp.int32),
            pltpu.VMEM((PRP // 128, 128), jnp.int32),
            pltpu.VMEM_SHARED((PRP, 128), jnp.int32),
        ],
    )(dst3, et3)


def _scales_body(src_hbm, dst_hbm, et_hbm, invc_hbm, gk_hbm, s_hbm,
                 gk_v, dst_v, et_v, s_v, invc_v):
    cid = lax.axis_index("c")
    sid = lax.axis_index("s")
    wid = sid * NC + cid
    pltpu.sync_copy(src_hbm.at[pl.ds(wid * NCH2, NCH2)], gk_v)
    pltpu.sync_copy(dst_hbm.at[pl.ds(wid * NCH2, NCH2)], dst_v)
    pltpu.sync_copy(et_hbm.at[pl.ds(wid * NCH2, NCH2)], et_v)
    pltpu.sync_copy(invc_hbm, invc_v)

    def keys_scales(g, _):
        j = g // 16
        l = (g - j * 16) * 16
        sl = pl.ds(l, 16)
        etv = et_v[j, sl] * NP
        sk = etv + dst_v[j, sl]
        gk_v[j, sl] = gk_v[j, sl] + etv
        s_v[j, sl] = plsc.load_gather(invc_v, [sk])
        return 0
    lax.fori_loop(0, EPW // 16, keys_scales, 0)
    pltpu.sync_copy(gk_v, gk_hbm.at[pl.ds(wid * NCH2, NCH2)])
    pltpu.sync_copy(s_v, s_hbm.at[pl.ds(wid * NCH2, NCH2)])


def _scales_call(src3, dst3, et3, invc):
    return pl.kernel(
        _scales_body,
        out_type=(
            jax.ShapeDtypeStruct((NW * NCH2, CH2), jnp.int32),
            jax.ShapeDtypeStruct((NW * NCH2, CH2), jnp.float32),
        ),
        mesh=_mesh(),
        compiler_params=_sc_params,
        scratch_types=[
            pltpu.VMEM((NCH2, CH2), jnp.int32),
            pltpu.VMEM((NCH2, CH2), jnp.int32),
            pltpu.VMEM((NCH2, CH2), jnp.int32),
            pltpu.VMEM((NCH2, CH2), jnp.float32),
            pltpu.VMEM((P,), jnp.float32),
        ],
    )(src3, dst3, et3, invc)


def _edge_body(t_hbm, gk_hbm, dst_hbm, s_hbm, acc_hbm,
               gk_v, dst_v, s_v, bufs, gsems, ssems, acc_sh):
    cid = lax.axis_index("c")
    sid = lax.axis_index("s")
    wid = sid * NC + cid
    pltpu.sync_copy(gk_hbm.at[pl.ds(wid * NCH2, NCH2)], gk_v)
    pltpu.sync_copy(dst_hbm.at[pl.ds(wid * NCH2, NCH2)], dst_v)
    pltpu.sync_copy(s_hbm.at[pl.ds(wid * NCH2, NCH2)], s_v)

    zv = jnp.zeros((16,), jnp.float32)
    rows0 = bufs[0]

    def zero_rows(g, _):
        r = g // 4
        c = g - r * 4
        rows0[r, pl.ds(c * 16, 16)] = zv
        return 0
    lax.fori_loop(0, CH * 4, zero_rows, 0)

    def zero_acc(t, _):
        pltpu.sync_copy(rows0.at[pl.ds(0, CH)],
                        acc_sh.at[pl.ds(sid * (NP // NS) + t * CH, CH)])
        return 0
    lax.fori_loop(0, NP // NS // CH, zero_acc, 0)
    plsc.subcore_barrier()

    def scale_chunk(rv, j):
        def scale16(eb, _):
            svec = s_v[j, pl.ds(eb * 16, 16)]
            for i in range(16):
                sval = svec[i]
                e = eb * 16 + i
                for c in range(4):
                    sl = pl.ds(c * 16, 16)
                    rv[e, sl] = rv[e, sl] * sval
            return 0
        lax.fori_loop(0, CH2 // 16, scale16, 0)

    def gidx(j):
        return gk_v.at[j]

    def sidx(j):
        return dst_v.at[j]

    pltpu.async_copy(t_hbm.at[gidx(0)], bufs[0], gsems[0])

    def chunk3(t, _):
        for p in range(3):
            j = t * 3 + p

            @pl.when(j < NCH2)
            def _():
                nb = (p + 1) % 3

                @pl.when(j >= 2)
                def _():
                    pltpu.make_async_copy(bufs[nb], acc_sh.at[sidx(j - 2)],
                                          ssems[nb]).wait()

                @pl.when(j + 1 < NCH2)
                def _():
                    pltpu.async_copy(t_hbm.at[gidx(j + 1)], bufs[nb],
                                     gsems[nb])
                pltpu.make_async_copy(t_hbm.at[gidx(j)], bufs[p],
                                      gsems[p]).wait()
                scale_chunk(bufs[p], j)
                pltpu.async_copy(bufs[p], acc_sh.at[sidx(j)], ssems[p],
                                 add=True)
        return 0
    lax.fori_loop(0, (NCH2 + 2) // 3, chunk3, 0)
    pltpu.make_async_copy(bufs[(NCH2 - 2) % 3], acc_sh.at[sidx(NCH2 - 2)],
                          ssems[(NCH2 - 2) % 3]).wait()
    pltpu.make_async_copy(bufs[(NCH2 - 1) % 3], acc_sh.at[sidx(NCH2 - 1)],
                          ssems[(NCH2 - 1) % 3]).wait()
    plsc.subcore_barrier()

    def export(t, _):
        base = sid * (NP // NS) + t * CH
        pltpu.sync_copy(acc_sh.at[pl.ds(base, CH)], rows0.at[pl.ds(0, CH)])
        pltpu.sync_copy(rows0.at[pl.ds(0, CH)],
                        acc_hbm.at[pl.ds(cid * NP + base, CH)])
        return 0
    lax.fori_loop(0, NP // NS // CH, export, 0)


def _edge_call(t_tab, gk3, dst3, s3):
    return pl.kernel(
        _edge_body,
        out_type=jax.ShapeDtypeStruct((NC * NP, D), jnp.float32),
        mesh=_mesh(),
        compiler_params=_sc_params,
        scratch_types=[
            pltpu.VMEM((NCH2, CH2), jnp.int32),
            pltpu.VMEM((NCH2, CH2), jnp.int32),
            pltpu.VMEM((NCH2, CH2), jnp.float32),
            [pltpu.VMEM((CH2, D), jnp.float32) for _ in range(3)],
            [pltpu.SemaphoreType.DMA for _ in range(3)],
            [pltpu.SemaphoreType.DMA for _ in range(3)],
            pltpu.VMEM_SHARED((NP, D), jnp.float32),
        ],
    )(t_tab, gk3, dst3, s3)


def _head_gather_body(idx_hbm, root_hbm, a0_hbm, a1_hbm, out_hbm,
                      idx_v, r0, r1, r2, sem):
    cid = lax.axis_index("c")
    sid = lax.axis_index("s")
    wid = sid * NC + cid
    pltpu.sync_copy(idx_hbm.at[pl.ds(wid * 3, 3)], idx_v)

    def chunk(j, _):
        pltpu.async_copy(root_hbm.at[idx_v.at[j]], r0, sem).wait()
        pltpu.async_copy(a0_hbm.at[idx_v.at[j]], r1, sem).wait()
        pltpu.async_copy(a1_hbm.at[idx_v.at[j]], r2, sem).wait()

        def addr(r, _):
            for c in range(4):
                sl = pl.ds(c * 16, 16)
                r0[r, sl] = r0[r, sl] + r1[r, sl] + r2[r, sl]
            return 0
        lax.fori_loop(0, CH, addr, 0)
        pltpu.sync_copy(r0, out_hbm.at[pl.ds(wid * 384 + j * CH, CH)])
        return 0
    lax.fori_loop(0, 3, chunk, 0)


def _head_gather_call(idx3, root2, a0, a1):
    return pl.kernel(
        _head_gather_body,
        out_type=jax.ShapeDtypeStruct((3 * B, D), jnp.float32),
        mesh=_mesh(),
        compiler_params=_sc_params,
        scratch_types=[
            pltpu.VMEM((3, CH), jnp.int32),
            pltpu.VMEM((CH, D), jnp.float32),
            pltpu.VMEM((CH, D), jnp.float32),
            pltpu.VMEM((CH, D), jnp.float32),
            pltpu.SemaphoreType.DMA,
        ],
    )(idx3, root2, a0, a1)


def _prep1_body(x_ref, w_ref, wr_ref, b_ref, cnt_ref, t_ref, root_ref,
                invc_ref):
    x = x_ref[...]
    t_ref[0:N, :] = jnp.dot(x, w_ref[0], preferred_element_type=jnp.float32)
    t_ref[N:NP, :] = jnp.zeros((NP - N, D), jnp.float32)
    root_ref[...] = (
        jnp.dot(x, wr_ref[...], preferred_element_type=jnp.float32)
        + b_ref[...]
    )
    cnt = cnt_ref[...]
    c = (cnt[0] + cnt[1]).astype(jnp.float32)
    rows = P // 128
    k = (lax.broadcasted_iota(jnp.int32, (rows, 128), 0) * 128
         + lax.broadcasted_iota(jnp.int32, (rows, 128), 1))
    npart = k % NP
    invc_ref[...] = jnp.where(npart < N, 1.0 / jnp.maximum(c, 1.0), 0.0)


def _prep1_call(x, w1_rel, w1_root, b1r, cnt2):
    rows = P // 128
    return pl.pallas_call(
        _prep1_body,
        grid=(R,),
        in_specs=[
            pl.BlockSpec((N, D_IN), lambda r: (0, 0)),
            pl.BlockSpec((1, D_IN, D), lambda r: (r, 0, 0)),
            pl.BlockSpec((D_IN, D), lambda r: (0, 0)),
            pl.BlockSpec((1, D), lambda r: (0, 0)),
            pl.BlockSpec((2, rows, 128), lambda r: (0, 0, 0)),
        ],
        out_specs=[
            pl.BlockSpec((NP, D), lambda r: (r, 0)),
            pl.BlockSpec((N, D), lambda r: (0, 0)),
            pl.BlockSpec((rows, 128), lambda r: (0, 0)),
        ],
        out_shape=[
            jax.ShapeDtypeStruct((P, D), jnp.float32),
            jax.ShapeDtypeStruct((N, D), jnp.float32),
            jax.ShapeDtypeStruct((rows, 128), jnp.float32),
        ],
    )(x, w1_rel, w1_root, b1r, cnt2)


def _mid_body(root1_ref, acc_ref, w_ref, wr_ref, b_ref, t_ref, root2_ref):
    a = acc_ref[...]
    h = root1_ref[...] + a[0:N, :] + a[NP:NP + N, :]
    h = jnp.maximum(h, 0.0)
    t_ref[0:N, :] = jnp.dot(h, w_ref[0], preferred_element_type=jnp.float32)
    t_ref[N:NP, :] = jnp.zeros((NP - N, D), jnp.float32)
    root2_ref[...] = (
        jnp.dot(h, wr_ref[...], preferred_element_type=jnp.float32)
        + b_ref[...]
    )


def _mid_call(root1, acc1, w2_rel, w2_root, b2r):
    return pl.pallas_call(
        _mid_body,
        grid=(R,),
        in_specs=[
            pl.BlockSpec((N, D), lambda r: (0, 0)),
            pl.BlockSpec((NC * NP, D), lambda r: (0, 0)),
            pl.BlockSpec((1, D, D), lambda r: (r, 0, 0)),
            pl.BlockSpec((D, D), lambda r: (0, 0)),
            pl.BlockSpec((1, D), lambda r: (0, 0)),
        ],
        out_specs=[
            pl.BlockSpec((NP, D), lambda r: (r, 0)),
            pl.BlockSpec((N, D), lambda r: (0, 0)),
        ],
        out_shape=[
            jax.ShapeDtypeStruct((P, D), jnp.float32),
            jax.ShapeDtypeStruct((N, D), jnp.float32),
        ],
    )(root1, acc1, w2_rel, w2_root, b2r)


def _head_body(rows_ref, w1_ref, b1_ref, w2t_ref, b2_ref, out_ref):
    xr = rows_ref[...]
    bill = xr[0:B]
    u1 = xr[B:2 * B]
    u2 = xr[2 * B:3 * B]
    w2b = jnp.broadcast_to(w2t_ref[...], (D, 128))
    b2v = b2_ref[...]

    def mlp(a, b):
        xcat = jnp.concatenate([a, b], axis=1)
        h1 = lax.dot_general(xcat, w1_ref[...], (((1,), (1,)), ((), ())),
                             preferred_element_type=jnp.float32)
        h1 = jnp.maximum(h1 + b1_ref[...], 0.0)
        return jnp.dot(h1, w2b, preferred_element_type=jnp.float32) + b2v[0, 0]

    p = mlp(bill, u1)
    q = mlp(bill, u2)

    def softplus(v):
        return jnp.maximum(v, 0.0) + jnp.log(1.0 + jnp.exp(-jnp.abs(v)))

    v = (softplus(p) - p) + softplus(q)
    s = jnp.sum(v, axis=0, keepdims=True)
    out_ref[...] = s[0:1, 0:1] / (2.0 * B)


def _head_call(rows3, fc1_w, fc1_br, fc2_wt, fc2_br):
    return pl.pallas_call(
        _head_body,
        out_shape=jax.ShapeDtypeStruct((1, 1), jnp.float32),
    )(rows3, fc1_w, fc1_br, fc2_wt, fc2_br)


def kernel(node_embeddings, w1_rel, w1_root, b1, w2_rel, w2_root, b2,
           fc1_w, fc1_b, fc2_w, fc2_b,
           edge_index_combined, edge_type_combined,
           bill_id, user1_id, user2_id):
    src = edge_index_combined[0].astype(jnp.int32)
    dst = edge_index_combined[1].astype(jnp.int32)
    et = edge_type_combined.astype(jnp.int32)
    pad = EP - E
    src_p = jnp.concatenate([src, jnp.zeros((pad,), jnp.int32)])
    dst_p = jnp.concatenate([dst, jnp.full((pad,), N, jnp.int32)])
    et_p = jnp.concatenate([et, jnp.zeros((pad,), jnp.int32)])
    src3 = src_p.reshape(NW * NCH2, CH2)
    dst3 = dst_p.reshape(NW * NCH2, CH2)
    et3 = et_p.reshape(NW * NCH2, CH2)

    cntp = _counts_call(dst3, et3)
    cnt2 = cntp.reshape(2, P // 128, 128)

    t1, root1, invc2 = _prep1_call(
        node_embeddings, w1_rel, w1_root, b1.reshape(1, D), cnt2)
    invc = invc2.reshape(P)

    gk3, s3 = _scales_call(src3, dst3, et3, invc)
    acc1 = _edge_call(t1, gk3, dst3, s3)
    t2, root2 = _mid_call(root1, acc1, w2_rel, w2_root, b2.reshape(1, D))
    acc2 = _edge_call(t2, gk3, dst3, s3)

    idx3 = jnp.concatenate([
        bill_id.astype(jnp.int32),
        user1_id.astype(jnp.int32),
        user2_id.astype(jnp.int32),
    ]).reshape(NW * 3, CH)
    rows3 = _head_gather_call(idx3, root2, acc2[0:NP], acc2[NP:])

    loss = _head_call(rows3, fc1_w, fc1_b.reshape(1, D),
                      fc2_w.reshape(D, 1), fc2_b.reshape(1, 1))
    return jnp.reshape(loss, ())


# speedup vs baseline: 12.4584x; 1.0790x over previous
import functools

import jax
import jax.numpy as jnp
from jax import lax
from jax.experimental import pallas as pl
from jax.experimental.pallas import tpu as pltpu
from jax.experimental.pallas import tpu_sc as plsc

N = 10000
NP = 10240
R = 4
P = R * NP
D_IN = 128
D = 64
E = 320000
B = 4096
NC = 2
NS = 16
NW = NC * NS
EPW = 10240
EP = NW * EPW
CH = 128
NCHUNK = EPW // CH
CH2 = 256
NCH2 = EPW // CH2
PR = P // 128
PRP = 384

_mesh = functools.partial(
    plsc.VectorSubcoreMesh,
    core_axis_name="c", subcore_axis_name="s",
    num_cores=NC, num_subcores=NS,
)
_sc_params = pltpu.CompilerParams(use_tc_tiling_on_sc=False,
                                  needs_layout_passes=False)


def _counts_body(dst_hbm, et_hbm, out_hbm, dst_v, et_v, cv, ibuf, csh):
    cid = lax.axis_index("c")
    sid = lax.axis_index("s")
    wid = sid * NC + cid
    pltpu.sync_copy(dst_hbm.at[pl.ds(wid * NCH2, NCH2)], dst_v)
    pltpu.sync_copy(et_hbm.at[pl.ds(wid * NCH2, NCH2)], et_v)

    z16 = jnp.zeros((16,), jnp.int32)

    def zero_cv(g, _):
        r = g // 8
        c = g - r * 8
        cv[r, pl.ds(c * 16, 16)] = z16
        return 0
    lax.fori_loop(0, PRP * 8, zero_cv, 0)

    def mk_iota(g, _):
        j = g // 8
        l = g - j * 8
        ibuf[j, pl.ds(l * 16, 16)] = lax.iota(jnp.int32, 16) + g * 16
        return 0
    lax.fori_loop(0, (PRP // 128) * 8, mk_iota, 0)

    pltpu.sync_copy(cv.at[pl.ds(sid * (PRP // NS), PRP // NS)],
                    csh.at[pl.ds(sid * (PRP // NS), PRP // NS)])
    plsc.subcore_barrier()

    one = jnp.ones((16,), jnp.int32)

    def count16(g, _):
        j = g // 16
        l = (g - j * 16) * 16
        sl = pl.ds(l, 16)
        kvec = et_v[j, sl] * NP + dst_v[j, sl]
        r = kvec // 128
        plsc.addupdate_scatter(cv, [r, kvec - r * 128], one)
        return 0
    lax.fori_loop(0, EPW // 16, count16, 0)

    def add_chunk(j, _):
        pltpu.sync_copy(cv.at[pl.ds(j * 128, 128)], csh.at[ibuf.at[j]],
                        add=True)
        return 0
    lax.fori_loop(0, PRP // 128, add_chunk, 0)
    plsc.subcore_barrier()

    base = sid * (PR // NS)
    pltpu.sync_copy(csh.at[pl.ds(base, PR // NS)], cv.at[pl.ds(0, PR // NS)])
    pltpu.sync_copy(cv.at[pl.ds(0, PR // NS)],
                    out_hbm.at[pl.ds(cid * PR + base, PR // NS)])


def _counts_call(dst3, et3):
    return pl.kernel(
        _counts_body,
        out_type=jax.ShapeDtypeStruct((NC * PR, 128), jnp.int32),
        mesh=_mesh(),
        compiler_params=_sc_params,
        scratch_types=[
            pltpu.VMEM((NCH2, CH2), jnp.int32),
            pltpu.VMEM((NCH2, CH2), jnp.int32),
            pltpu.VMEM((PRP, 128), jnp.int32),
            pltpu.VMEM((PRP // 128, 128), jnp.int32),
            pltpu.VMEM_SHARED((PRP, 128), jnp.int32),
        ],
    )(dst3, et3)


def _scales_body(src_hbm, dst_hbm, et_hbm, invc_hbm, gk_hbm, s_hbm,
                 gk_v, dst_v, et_v, s_v, invc_v):
    cid = lax.axis_index("c")
    sid = lax.axis_index("s")
    wid = sid * NC + cid
    pltpu.sync_copy(src_hbm.at[pl.ds(wid * NCH2, NCH2)], gk_v)
    pltpu.sync_copy(dst_hbm.at[pl.ds(wid * NCH2, NCH2)], dst_v)
    pltpu.sync_copy(et_hbm.at[pl.ds(wid * NCH2, NCH2)], et_v)
    pltpu.sync_copy(invc_hbm, invc_v)

    def keys_scales(g, _):
        j = g // 16
        l = (g - j * 16) * 16
        sl = pl.ds(l, 16)
        etv = et_v[j, sl] * NP
        sk = etv + dst_v[j, sl]
        gk_v[j, sl] = gk_v[j, sl] + etv
        skr = sk // 128
        s_v[j, sl] = plsc.load_gather(invc_v, [skr, sk - skr * 128])
        return 0
    lax.fori_loop(0, EPW // 16, keys_scales, 0)
    pltpu.sync_copy(gk_v, gk_hbm.at[pl.ds(wid * NCH2, NCH2)])
    pltpu.sync_copy(s_v, s_hbm.at[pl.ds(wid * NCH2, NCH2)])


def _scales_call(src3, dst3, et3, invc):
    return pl.kernel(
        _scales_body,
        out_type=(
            jax.ShapeDtypeStruct((NW * NCH2, CH2), jnp.int32),
            jax.ShapeDtypeStruct((NW * NCH2, CH2), jnp.float32),
        ),
        mesh=_mesh(),
        compiler_params=_sc_params,
        scratch_types=[
            pltpu.VMEM((NCH2, CH2), jnp.int32),
            pltpu.VMEM((NCH2, CH2), jnp.int32),
            pltpu.VMEM((NCH2, CH2), jnp.int32),
            pltpu.VMEM((NCH2, CH2), jnp.float32),
            pltpu.VMEM((PR, 128), jnp.float32),
        ],
    )(src3, dst3, et3, invc)


def _edge_body(t_hbm, gk_hbm, dst_hbm, s_hbm, acc_hbm,
               gk_v, dst_v, s_v, bufs, gsems, ssems, acc_sh):
    cid = lax.axis_index("c")
    sid = lax.axis_index("s")
    wid = sid * NC + cid
    pltpu.sync_copy(gk_hbm.at[pl.ds(wid * NCH2, NCH2)], gk_v)
    pltpu.sync_copy(dst_hbm.at[pl.ds(wid * NCH2, NCH2)], dst_v)
    pltpu.sync_copy(s_hbm.at[pl.ds(wid * NCH2, NCH2)], s_v)

    zv = jnp.zeros((16,), jnp.float32)
    rows0 = bufs[0]

    def zero_rows(g, _):
        r = g // 4
        c = g - r * 4
        rows0[r, pl.ds(c * 16, 16)] = zv
        return 0
    lax.fori_loop(0, CH * 4, zero_rows, 0)

    def zero_acc(t, _):
        pltpu.sync_copy(rows0.at[pl.ds(0, CH)],
                        acc_sh.at[pl.ds(sid * (NP // NS) + t * CH, CH)])
        return 0
    lax.fori_loop(0, NP // NS // CH, zero_acc, 0)
    plsc.subcore_barrier()

    def scale_chunk(rv, j):
        def scale16(eb, _):
            svec = s_v[j, pl.ds(eb * 16, 16)]
            for i in range(16):
                sval = svec[i]
                e = eb * 16 + i
                for c in range(4):
                    sl = pl.ds(c * 16, 16)
                    rv[e, sl] = rv[e, sl] * sval
            return 0
        lax.fori_loop(0, CH2 // 16, scale16, 0)

    def gidx(j):
        return gk_v.at[j]

    def sidx(j):
        return dst_v.at[j]

    pltpu.async_copy(t_hbm.at[gidx(0)], bufs[0], gsems[0])

    def chunk3(t, _):
        for p in range(3):
            j = t * 3 + p

            @pl.when(j < NCH2)
            def _():
                nb = (p + 1) % 3

                @pl.when(j >= 2)
                def _():
                    pltpu.make_async_copy(bufs[nb], acc_sh.at[sidx(j - 2)],
                                          ssems[nb]).wait()

                @pl.when(j + 1 < NCH2)
                def _():
                    pltpu.async_copy(t_hbm.at[gidx(j + 1)], bufs[nb],
                                     gsems[nb])
                pltpu.make_async_copy(t_hbm.at[gidx(j)], bufs[p],
                                      gsems[p]).wait()
                scale_chunk(bufs[p], j)
                pltpu.async_copy(bufs[p], acc_sh.at[sidx(j)], ssems[p],
                                 add=True)
        return 0
    lax.fori_loop(0, (NCH2 + 2) // 3, chunk3, 0)
    pltpu.make_async_copy(bufs[(NCH2 - 2) % 3], acc_sh.at[sidx(NCH2 - 2)],
                          ssems[(NCH2 - 2) % 3]).wait()
    pltpu.make_async_copy(bufs[(NCH2 - 1) % 3], acc_sh.at[sidx(NCH2 - 1)],
                          ssems[(NCH2 - 1) % 3]).wait()
    plsc.subcore_barrier()

    def export(t, _):
        base = sid * (NP // NS) + t * CH
        pltpu.sync_copy(acc_sh.at[pl.ds(base, CH)], rows0.at[pl.ds(0, CH)])
        pltpu.sync_copy(rows0.at[pl.ds(0, CH)],
                        acc_hbm.at[pl.ds(cid * NP + base, CH)])
        return 0
    lax.fori_loop(0, NP // NS // CH, export, 0)


def _edge_call(t_tab, gk3, dst3, s3):
    return pl.kernel(
        _edge_body,
        out_type=jax.ShapeDtypeStruct((NC * NP, D), jnp.float32),
        mesh=_mesh(),
        compiler_params=_sc_params,
        scratch_types=[
            pltpu.VMEM((NCH2, CH2), jnp.int32),
            pltpu.VMEM((NCH2, CH2), jnp.int32),
            pltpu.VMEM((NCH2, CH2), jnp.float32),
            [pltpu.VMEM((CH2, D), jnp.float32) for _ in range(3)],
            [pltpu.SemaphoreType.DMA for _ in range(3)],
            [pltpu.SemaphoreType.DMA for _ in range(3)],
            pltpu.VMEM_SHARED((NP, D), jnp.float32),
        ],
    )(t_tab, gk3, dst3, s3)


def _head_gather_body(idx_hbm, root_hbm, a0_hbm, a1_hbm, out_hbm,
                      idx_v, r0, r1, r2, sem):
    cid = lax.axis_index("c")
    sid = lax.axis_index("s")
    wid = sid * NC + cid
    pltpu.sync_copy(idx_hbm.at[pl.ds(wid * 3, 3)], idx_v)

    def chunk(j, _):
        pltpu.async_copy(root_hbm.at[idx_v.at[j]], r0, sem).wait()
        pltpu.async_copy(a0_hbm.at[idx_v.at[j]], r1, sem).wait()
        pltpu.async_copy(a1_hbm.at[idx_v.at[j]], r2, sem).wait()

        def addr(r, _):
            for c in range(4):
                sl = pl.ds(c * 16, 16)
                r0[r, sl] = r0[r, sl] + r1[r, sl] + r2[r, sl]
            return 0
        lax.fori_loop(0, CH, addr, 0)
        pltpu.sync_copy(r0, out_hbm.at[pl.ds(wid * 384 + j * CH, CH)])
        return 0
    lax.fori_loop(0, 3, chunk, 0)


def _head_gather_call(idx3, root2, a0, a1):
    return pl.kernel(
        _head_gather_body,
        out_type=jax.ShapeDtypeStruct((3 * B, D), jnp.float32),
        mesh=_mesh(),
        compiler_params=_sc_params,
        scratch_types=[
            pltpu.VMEM((3, CH), jnp.int32),
            pltpu.VMEM((CH, D), jnp.float32),
            pltpu.VMEM((CH, D), jnp.float32),
            pltpu.VMEM((CH, D), jnp.float32),
            pltpu.SemaphoreType.DMA,
        ],
    )(idx3, root2, a0, a1)


def _prep1_body(x_ref, w_ref, wr_ref, b_ref, cnt_ref, t_ref, root_ref,
                invc_ref):
    x = x_ref[...]
    t_ref[0:N, :] = jnp.dot(x, w_ref[0], preferred_element_type=jnp.float32)
    t_ref[N:NP, :] = jnp.zeros((NP - N, D), jnp.float32)
    root_ref[...] = (
        jnp.dot(x, wr_ref[...], preferred_element_type=jnp.float32)
        + b_ref[...]
    )
    cnt = cnt_ref[...]
    c = (cnt[0:PR] + cnt[PR:2 * PR]).astype(jnp.float32)
    rows = P // 128
    k = (lax.broadcasted_iota(jnp.int32, (rows, 128), 0) * 128
         + lax.broadcasted_iota(jnp.int32, (rows, 128), 1))
    npart = k % NP
    invc_ref[...] = jnp.where(npart < N, 1.0 / jnp.maximum(c, 1.0), 0.0)


def _prep1_call(x, w1_rel, w1_root, b1r, cnt2):
    rows = P // 128
    return pl.pallas_call(
        _prep1_body,
        grid=(R,),
        in_specs=[
            pl.BlockSpec((N, D_IN), lambda r: (0, 0)),
            pl.BlockSpec((1, D_IN, D), lambda r: (r, 0, 0)),
            pl.BlockSpec((D_IN, D), lambda r: (0, 0)),
            pl.BlockSpec((1, D), lambda r: (0, 0)),
            pl.BlockSpec((2 * PR, 128), lambda r: (0, 0)),
        ],
        out_specs=[
            pl.BlockSpec((NP, D), lambda r: (r, 0)),
            pl.BlockSpec((N, D), lambda r: (0, 0)),
            pl.BlockSpec((rows, 128), lambda r: (0, 0)),
        ],
        out_shape=[
            jax.ShapeDtypeStruct((P, D), jnp.float32),
            jax.ShapeDtypeStruct((N, D), jnp.float32),
            jax.ShapeDtypeStruct((rows, 128), jnp.float32),
        ],
    )(x, w1_rel, w1_root, b1r, cnt2)


def _mid_body(root1_ref, acc_ref, w_ref, wr_ref, b_ref, t_ref, root2_ref):
    a = acc_ref[...]
    h = root1_ref[...] + a[0:N, :] + a[NP:NP + N, :]
    h = jnp.maximum(h, 0.0)
    t_ref[0:N, :] = jnp.dot(h, w_ref[0], preferred_element_type=jnp.float32)
    t_ref[N:NP, :] = jnp.zeros((NP - N, D), jnp.float32)
    root2_ref[...] = (
        jnp.dot(h, wr_ref[...], preferred_element_type=jnp.float32)
        + b_ref[...]
    )


def _mid_call(root1, acc1, w2_rel, w2_root, b2r):
    return pl.pallas_call(
        _mid_body,
        grid=(R,),
        in_specs=[
            pl.BlockSpec((N, D), lambda r: (0, 0)),
            pl.BlockSpec((NC * NP, D), lambda r: (0, 0)),
            pl.BlockSpec((1, D, D), lambda r: (r, 0, 0)),
            pl.BlockSpec((D, D), lambda r: (0, 0)),
            pl.BlockSpec((1, D), lambda r: (0, 0)),
        ],
        out_specs=[
            pl.BlockSpec((NP, D), lambda r: (r, 0)),
            pl.BlockSpec((N, D), lambda r: (0, 0)),
        ],
        out_shape=[
            jax.ShapeDtypeStruct((P, D), jnp.float32),
            jax.ShapeDtypeStruct((N, D), jnp.float32),
        ],
    )(root1, acc1, w2_rel, w2_root, b2r)


def _head_body(rows_ref, w1_ref, b1_ref, w2t_ref, b2_ref, out_ref):
    xr = rows_ref[...]
    bill = xr[0:B]
    u1 = xr[B:2 * B]
    u2 = xr[2 * B:3 * B]
    w2b = jnp.broadcast_to(w2t_ref[...], (D, 128))
    b2v = b2_ref[...]

    def mlp(a, b):
        xcat = jnp.concatenate([a, b], axis=1)
        h1 = lax.dot_general(xcat, w1_ref[...], (((1,), (1,)), ((), ())),
                             preferred_element_type=jnp.float32)
        h1 = jnp.maximum(h1 + b1_ref[...], 0.0)
        return jnp.dot(h1, w2b, preferred_element_type=jnp.float32) + b2v[0, 0]

    p = mlp(bill, u1)
    q = mlp(bill, u2)

    def softplus(v):
        return jnp.maximum(v, 0.0) + jnp.log(1.0 + jnp.exp(-jnp.abs(v)))

    v = (softplus(p) - p) + softplus(q)
    s = jnp.sum(v, axis=0, keepdims=True)
    out_ref[...] = s[0:1, 0:1] / (2.0 * B)


def _head_call(rows3, fc1_w, fc1_br, fc2_wt, fc2_br):
    return pl.pallas_call(
        _head_body,
        out_shape=jax.ShapeDtypeStruct((1, 1), jnp.float32),
    )(rows3, fc1_w, fc1_br, fc2_wt, fc2_br)


def kernel(node_embeddings, w1_rel, w1_root, b1, w2_rel, w2_root, b2,
           fc1_w, fc1_b, fc2_w, fc2_b,
           edge_index_combined, edge_type_combined,
           bill_id, user1_id, user2_id):
    src = edge_index_combined[0].astype(jnp.int32)
    dst = edge_index_combined[1].astype(jnp.int32)
    et = edge_type_combined.astype(jnp.int32)
    pad = EP - E
    src_p = jnp.concatenate([src, jnp.zeros((pad,), jnp.int32)])
    dst_p = jnp.concatenate([dst, jnp.full((pad,), N, jnp.int32)])
    et_p = jnp.concatenate([et, jnp.zeros((pad,), jnp.int32)])
    src3 = src_p.reshape(NW * NCH2, CH2)
    dst3 = dst_p.reshape(NW * NCH2, CH2)
    et3 = et_p.reshape(NW * NCH2, CH2)

    cntp = _counts_call(dst3, et3)

    t1, root1, invc2 = _prep1_call(
        node_embeddings, w1_rel, w1_root, b1.reshape(1, D), cntp)

    gk3, s3 = _scales_call(src3, dst3, et3, invc2)
    acc1 = _edge_call(t1, gk3, dst3, s3)
    t2, root2 = _mid_call(root1, acc1, w2_rel, w2_root, b2.reshape(1, D))
    acc2 = _edge_call(t2, gk3, dst3, s3)

    idx3 = jnp.concatenate([
        bill_id.astype(jnp.int32),
        user1_id.astype(jnp.int32),
        user2_id.astype(jnp.int32),
    ]).reshape(NW * 3, CH)
    rows3 = _head_gather_call(idx3, root2, acc2[0:NP], acc2[NP:])

    loss = _head_call(rows3, fc1_w, fc1_b.reshape(1, D),
                      fc2_w.reshape(D, 1), fc2_b.reshape(1, 1))
    return jnp.reshape(loss, ())


# speedup vs baseline: 13.2411x; 1.0628x over previous
import functools

import jax
import jax.numpy as jnp
from jax import lax
from jax.experimental import pallas as pl
from jax.experimental.pallas import tpu as pltpu
from jax.experimental.pallas import tpu_sc as plsc

N = 10000
NP = 10240
R = 4
P = R * NP
D_IN = 128
D = 64
E = 320000
B = 4096
NC = 2
NS = 16
NW = NC * NS
EPW = 10240
EP = NW * EPW
CH = 128
NCHUNK = EPW // CH
CH2 = 256
NCH2 = EPW // CH2
NCH_A = 48
NCH_B = 2 * NCH2 - NCH_A
EROWS = NW * NCH2
EROWS_PAD = NS * NCH_A + NS * NCH_B + (NCH_A - NCH_B)
PR = P // 128
PRP = 384

_mesh = functools.partial(
    plsc.VectorSubcoreMesh,
    core_axis_name="c", subcore_axis_name="s",
    num_cores=NC, num_subcores=NS,
)
_sc_params = pltpu.CompilerParams(use_tc_tiling_on_sc=False,
                                  needs_layout_passes=False)


def _counts_body(dst_hbm, et_hbm, out_hbm, dst_v, et_v, cv, ibuf, csh):
    cid = lax.axis_index("c")
    sid = lax.axis_index("s")
    wid = sid * NC + cid
    pltpu.sync_copy(dst_hbm.at[pl.ds(wid * NCH2, NCH2)], dst_v)
    pltpu.sync_copy(et_hbm.at[pl.ds(wid * NCH2, NCH2)], et_v)

    z16 = jnp.zeros((16,), jnp.int32)

    def zero_cv(g, _):
        r = g // 8
        c = g - r * 8
        cv[r, pl.ds(c * 16, 16)] = z16
        return 0
    lax.fori_loop(0, PRP * 8, zero_cv, 0)

    def mk_iota(g, _):
        j = g // 8
        l = g - j * 8
        ibuf[j, pl.ds(l * 16, 16)] = lax.iota(jnp.int32, 16) + g * 16
        return 0
    lax.fori_loop(0, (PRP // 128) * 8, mk_iota, 0)

    pltpu.sync_copy(cv.at[pl.ds(sid * (PRP // NS), PRP // NS)],
                    csh.at[pl.ds(sid * (PRP // NS), PRP // NS)])
    plsc.subcore_barrier()

    one = jnp.ones((16,), jnp.int32)

    def count16(g, _):
        j = g // 16
        l = (g - j * 16) * 16
        sl = pl.ds(l, 16)
        kvec = et_v[j, sl] * NP + dst_v[j, sl]
        r = kvec // 128
        plsc.addupdate_scatter(cv, [r, kvec - r * 128], one)
        return 0
    lax.fori_loop(0, EPW // 16, count16, 0)

    def add_chunk(j, _):
        pltpu.sync_copy(cv.at[pl.ds(j * 128, 128)], csh.at[ibuf.at[j]],
                        add=True)
        return 0
    lax.fori_loop(0, PRP // 128, add_chunk, 0)
    plsc.subcore_barrier()

    base = sid * (PR // NS)
    pltpu.sync_copy(csh.at[pl.ds(base, PR // NS)], cv.at[pl.ds(0, PR // NS)])
    pltpu.sync_copy(cv.at[pl.ds(0, PR // NS)],
                    out_hbm.at[pl.ds(cid * PR + base, PR // NS)])


def _counts_call(dst3, et3):
    return pl.kernel(
        _counts_body,
        out_type=jax.ShapeDtypeStruct((NC * PR, 128), jnp.int32),
        mesh=_mesh(),
        compiler_params=_sc_params,
        scratch_types=[
            pltpu.VMEM((NCH2, CH2), jnp.int32),
            pltpu.VMEM((NCH2, CH2), jnp.int32),
            pltpu.VMEM((PRP, 128), jnp.int32),
            pltpu.VMEM((PRP // 128, 128), jnp.int32),
            pltpu.VMEM_SHARED((PRP, 128), jnp.int32),
        ],
    )(dst3, et3)


def _scales_body(src_hbm, dst_hbm, et_hbm, invc_hbm, gk_hbm, s_hbm,
                 gk_v, dst_v, et_v, s_v, invc_v):
    cid = lax.axis_index("c")
    sid = lax.axis_index("s")
    wid = sid * NC + cid
    pltpu.sync_copy(src_hbm.at[pl.ds(wid * NCH2, NCH2)], gk_v)
    pltpu.sync_copy(dst_hbm.at[pl.ds(wid * NCH2, NCH2)], dst_v)
    pltpu.sync_copy(et_hbm.at[pl.ds(wid * NCH2, NCH2)], et_v)
    pltpu.sync_copy(invc_hbm, invc_v)

    def keys_scales(g, _):
        j = g // 16
        l = (g - j * 16) * 16
        sl = pl.ds(l, 16)
        etv = et_v[j, sl] * NP
        sk = etv + dst_v[j, sl]
        gk_v[j, sl] = gk_v[j, sl] + etv
        skr = sk // 128
        s_v[j, sl] = plsc.load_gather(invc_v, [skr, sk - skr * 128])
        return 0
    lax.fori_loop(0, EPW // 16, keys_scales, 0)
    pltpu.sync_copy(gk_v, gk_hbm.at[pl.ds(wid * NCH2, NCH2)])
    pltpu.sync_copy(s_v, s_hbm.at[pl.ds(wid * NCH2, NCH2)])


def _scales_call(src3, dst3, et3, invc):
    return pl.kernel(
        _scales_body,
        out_type=(
            jax.ShapeDtypeStruct((EROWS_PAD, CH2), jnp.int32),
            jax.ShapeDtypeStruct((EROWS_PAD, CH2), jnp.float32),
        ),
        mesh=_mesh(),
        compiler_params=_sc_params,
        scratch_types=[
            pltpu.VMEM((NCH2, CH2), jnp.int32),
            pltpu.VMEM((NCH2, CH2), jnp.int32),
            pltpu.VMEM((NCH2, CH2), jnp.int32),
            pltpu.VMEM((NCH2, CH2), jnp.float32),
            pltpu.VMEM((PR, 128), jnp.float32),
        ],
    )(src3, dst3, et3, invc)


def _edge_body(t_hbm, gk_hbm, dst_hbm, s_hbm, acc_hbm,
               gk_v, dst_v, s_v, bufs, gsems, ssems, acc_sh):
    cid = lax.axis_index("c")
    sid = lax.axis_index("s")
    base = jnp.where(cid == 0, sid * NCH_A, NS * NCH_A + sid * NCH_B)
    nch = jnp.where(cid == 0, NCH_A, NCH_B)
    pltpu.sync_copy(gk_hbm.at[pl.ds(base, NCH_A)], gk_v)
    pltpu.sync_copy(dst_hbm.at[pl.ds(base, NCH_A)], dst_v)
    pltpu.sync_copy(s_hbm.at[pl.ds(base, NCH_A)], s_v)

    zv = jnp.zeros((16,), jnp.float32)
    rows0 = bufs[0]

    def zero_rows(g, _):
        r = g // 4
        c = g - r * 4
        rows0[r, pl.ds(c * 16, 16)] = zv
        return 0
    lax.fori_loop(0, CH * 4, zero_rows, 0)

    def zero_acc(t, _):
        pltpu.sync_copy(rows0.at[pl.ds(0, CH)],
                        acc_sh.at[pl.ds(sid * (NP // NS) + t * CH, CH)])
        return 0
    lax.fori_loop(0, NP // NS // CH, zero_acc, 0)
    plsc.subcore_barrier()

    def scale_chunk(rv, j):
        def scale16(eb, _):
            svec = s_v[j, pl.ds(eb * 16, 16)]
            for i in range(16):
                sval = svec[i]
                e = eb * 16 + i
                for c in range(4):
                    sl = pl.ds(c * 16, 16)
                    rv[e, sl] = rv[e, sl] * sval
            return 0
        lax.fori_loop(0, CH2 // 16, scale16, 0)

    def gidx(j):
        return gk_v.at[j]

    def sidx(j):
        return dst_v.at[j]

    pltpu.async_copy(t_hbm.at[gidx(0)], bufs[0], gsems[0])

    def chunk3(t, _):
        for p in range(3):
            j = t * 3 + p

            @pl.when(j < nch)
            def _():
                nb = (p + 1) % 3

                @pl.when(j >= 2)
                def _():
                    pltpu.make_async_copy(bufs[nb], acc_sh.at[sidx(j - 2)],
                                          ssems[nb]).wait()

                @pl.when(j + 1 < nch)
                def _():
                    pltpu.async_copy(t_hbm.at[gidx(j + 1)], bufs[nb],
                                     gsems[nb])
                pltpu.make_async_copy(t_hbm.at[gidx(j)], bufs[p],
                                      gsems[p]).wait()
                scale_chunk(bufs[p], j)
                pltpu.async_copy(bufs[p], acc_sh.at[sidx(j)], ssems[p],
                                 add=True)
        return 0
    lax.fori_loop(0, (NCH_A + 2) // 3, chunk3, 0)

    @pl.when(cid == 0)
    def _():
        pltpu.make_async_copy(bufs[(NCH_A - 2) % 3], acc_sh.at[sidx(NCH_A - 2)],
                              ssems[(NCH_A - 2) % 3]).wait()
        pltpu.make_async_copy(bufs[(NCH_A - 1) % 3], acc_sh.at[sidx(NCH_A - 1)],
                              ssems[(NCH_A - 1) % 3]).wait()

    @pl.when(cid == 1)
    def _():
        pltpu.make_async_copy(bufs[(NCH_B - 2) % 3], acc_sh.at[sidx(NCH_B - 2)],
                              ssems[(NCH_B - 2) % 3]).wait()
        pltpu.make_async_copy(bufs[(NCH_B - 1) % 3], acc_sh.at[sidx(NCH_B - 1)],
                              ssems[(NCH_B - 1) % 3]).wait()
    plsc.subcore_barrier()

    def export(t, _):
        base = sid * (NP // NS) + t * CH
        pltpu.sync_copy(acc_sh.at[pl.ds(base, CH)], rows0.at[pl.ds(0, CH)])
        pltpu.sync_copy(rows0.at[pl.ds(0, CH)],
                        acc_hbm.at[pl.ds(cid * NP + base, CH)])
        return 0
    lax.fori_loop(0, NP // NS // CH, export, 0)


def _edge_call(t_tab, gk3, dst3, s3):
    return pl.kernel(
        _edge_body,
        out_type=jax.ShapeDtypeStruct((NC * NP, D), jnp.float32),
        mesh=_mesh(),
        compiler_params=_sc_params,
        scratch_types=[
            pltpu.VMEM((NCH_A, CH2), jnp.int32),
            pltpu.VMEM((NCH_A, CH2), jnp.int32),
            pltpu.VMEM((NCH_A, CH2), jnp.float32),
            [pltpu.VMEM((CH2, D), jnp.float32) for _ in range(3)],
            [pltpu.SemaphoreType.DMA for _ in range(3)],
            [pltpu.SemaphoreType.DMA for _ in range(3)],
            pltpu.VMEM_SHARED((NP, D), jnp.float32),
        ],
    )(t_tab, gk3, dst3, s3)


def _head_gather_body(idx_hbm, root_hbm, a0_hbm, a1_hbm, out_hbm,
                      idx_v, r0, r1, r2, sem):
    cid = lax.axis_index("c")
    sid = lax.axis_index("s")
    wid = sid * NC + cid
    pltpu.sync_copy(idx_hbm.at[pl.ds(wid * 3, 3)], idx_v)

    def chunk(j, _):
        pltpu.async_copy(root_hbm.at[idx_v.at[j]], r0, sem).wait()
        pltpu.async_copy(a0_hbm.at[idx_v.at[j]], r1, sem).wait()
        pltpu.async_copy(a1_hbm.at[idx_v.at[j]], r2, sem).wait()

        def addr(r, _):
            for c in range(4):
                sl = pl.ds(c * 16, 16)
                r0[r, sl] = r0[r, sl] + r1[r, sl] + r2[r, sl]
            return 0
        lax.fori_loop(0, CH, addr, 0)
        pltpu.sync_copy(r0, out_hbm.at[pl.ds(wid * 384 + j * CH, CH)])
        return 0
    lax.fori_loop(0, 3, chunk, 0)


def _head_gather_call(idx3, root2, a0, a1):
    return pl.kernel(
        _head_gather_body,
        out_type=jax.ShapeDtypeStruct((3 * B, D), jnp.float32),
        mesh=_mesh(),
        compiler_params=_sc_params,
        scratch_types=[
            pltpu.VMEM((3, CH), jnp.int32),
            pltpu.VMEM((CH, D), jnp.float32),
            pltpu.VMEM((CH, D), jnp.float32),
            pltpu.VMEM((CH, D), jnp.float32),
            pltpu.SemaphoreType.DMA,
        ],
    )(idx3, root2, a0, a1)


def _prep1_body(x_ref, w_ref, wr_ref, b_ref, cnt_ref, t_ref, root_ref,
                invc_ref):
    x = x_ref[...]
    t_ref[0:N, :] = jnp.dot(x, w_ref[0], preferred_element_type=jnp.float32)
    t_ref[N:NP, :] = jnp.zeros((NP - N, D), jnp.float32)
    root_ref[...] = (
        jnp.dot(x, wr_ref[...], preferred_element_type=jnp.float32)
        + b_ref[...]
    )
    cnt = cnt_ref[...]
    c = (cnt[0:PR] + cnt[PR:2 * PR]).astype(jnp.float32)
    rows = P // 128
    k = (lax.broadcasted_iota(jnp.int32, (rows, 128), 0) * 128
         + lax.broadcasted_iota(jnp.int32, (rows, 128), 1))
    npart = k % NP
    invc_ref[...] = jnp.where(npart < N, 1.0 / jnp.maximum(c, 1.0), 0.0)


def _prep1_call(x, w1_rel, w1_root, b1r, cnt2):
    rows = P // 128
    return pl.pallas_call(
        _prep1_body,
        grid=(R,),
        in_specs=[
            pl.BlockSpec((N, D_IN), lambda r: (0, 0)),
            pl.BlockSpec((1, D_IN, D), lambda r: (r, 0, 0)),
            pl.BlockSpec((D_IN, D), lambda r: (0, 0)),
            pl.BlockSpec((1, D), lambda r: (0, 0)),
            pl.BlockSpec((2 * PR, 128), lambda r: (0, 0)),
        ],
        out_specs=[
            pl.BlockSpec((NP, D), lambda r: (r, 0)),
            pl.BlockSpec((N, D), lambda r: (0, 0)),
            pl.BlockSpec((rows, 128), lambda r: (0, 0)),
        ],
        out_shape=[
            jax.ShapeDtypeStruct((P, D), jnp.float32),
            jax.ShapeDtypeStruct((N, D), jnp.float32),
            jax.ShapeDtypeStruct((rows, 128), jnp.float32),
        ],
    )(x, w1_rel, w1_root, b1r, cnt2)


def _mid_body(root1_ref, acc_ref, w_ref, wr_ref, b_ref, t_ref, root2_ref):
    a = acc_ref[...]
    h = root1_ref[...] + a[0:N, :] + a[NP:NP + N, :]
    h = jnp.maximum(h, 0.0)
    t_ref[0:N, :] = jnp.dot(h, w_ref[0], preferred_element_type=jnp.float32)
    t_ref[N:NP, :] = jnp.zeros((NP - N, D), jnp.float32)
    root2_ref[...] = (
        jnp.dot(h, wr_ref[...], preferred_element_type=jnp.float32)
        + b_ref[...]
    )


def _mid_call(root1, acc1, w2_rel, w2_root, b2r):
    return pl.pallas_call(
        _mid_body,
        grid=(R,),
        in_specs=[
            pl.BlockSpec((N, D), lambda r: (0, 0)),
            pl.BlockSpec((NC * NP, D), lambda r: (0, 0)),
            pl.BlockSpec((1, D, D), lambda r: (r, 0, 0)),
            pl.BlockSpec((D, D), lambda r: (0, 0)),
            pl.BlockSpec((1, D), lambda r: (0, 0)),
        ],
        out_specs=[
            pl.BlockSpec((NP, D), lambda r: (r, 0)),
            pl.BlockSpec((N, D), lambda r: (0, 0)),
        ],
        out_shape=[
            jax.ShapeDtypeStruct((P, D), jnp.float32),
            jax.ShapeDtypeStruct((N, D), jnp.float32),
        ],
    )(root1, acc1, w2_rel, w2_root, b2r)


def _head_body(rows_ref, w1_ref, b1_ref, w2t_ref, b2_ref, out_ref):
    xr = rows_ref[...]
    bill = xr[0:B]
    u1 = xr[B:2 * B]
    u2 = xr[2 * B:3 * B]
    w2b = jnp.broadcast_to(w2t_ref[...], (D, 128))
    b2v = b2_ref[...]

    def mlp(a, b):
        xcat = jnp.concatenate([a, b], axis=1)
        h1 = lax.dot_general(xcat, w1_ref[...], (((1,), (1,)), ((), ())),
                             preferred_element_type=jnp.float32)
        h1 = jnp.maximum(h1 + b1_ref[...], 0.0)
        return jnp.dot(h1, w2b, preferred_element_type=jnp.float32) + b2v[0, 0]

    p = mlp(bill, u1)
    q = mlp(bill, u2)

    def softplus(v):
        return jnp.maximum(v, 0.0) + jnp.log(1.0 + jnp.exp(-jnp.abs(v)))

    v = (softplus(p) - p) + softplus(q)
    s = jnp.sum(v, axis=0, keepdims=True)
    out_ref[...] = s[0:1, 0:1] / (2.0 * B)


def _head_call(rows3, fc1_w, fc1_br, fc2_wt, fc2_br):
    return pl.pallas_call(
        _head_body,
        out_shape=jax.ShapeDtypeStruct((1, 1), jnp.float32),
    )(rows3, fc1_w, fc1_br, fc2_wt, fc2_br)


def kernel(node_embeddings, w1_rel, w1_root, b1, w2_rel, w2_root, b2,
           fc1_w, fc1_b, fc2_w, fc2_b,
           edge_index_combined, edge_type_combined,
           bill_id, user1_id, user2_id):
    src = edge_index_combined[0].astype(jnp.int32)
    dst = edge_index_combined[1].astype(jnp.int32)
    et = edge_type_combined.astype(jnp.int32)
    pad = EROWS_PAD * CH2 - E
    src_p = jnp.concatenate([src, jnp.zeros((pad,), jnp.int32)])
    dst_p = jnp.concatenate([dst, jnp.full((pad,), N, jnp.int32)])
    et_p = jnp.concatenate([et, jnp.zeros((pad,), jnp.int32)])
    src3 = src_p.reshape(EROWS_PAD, CH2)
    dst3 = dst_p.reshape(EROWS_PAD, CH2)
    et3 = et_p.reshape(EROWS_PAD, CH2)

    cntp = _counts_call(dst3, et3)

    t1, root1, invc2 = _prep1_call(
        node_embeddings, w1_rel, w1_root, b1.reshape(1, D), cntp)

    gk3, s3 = _scales_call(src3, dst3, et3, invc2)
    acc1 = _edge_call(t1, gk3, dst3, s3)
    t2, root2 = _mid_call(root1, acc1, w2_rel, w2_root, b2.reshape(1, D))
    acc2 = _edge_call(t2, gk3, dst3, s3)

    idx3 = jnp.concatenate([
        bill_id.astype(jnp.int32),
        user1_id.astype(jnp.int32),
        user2_id.astype(jnp.int32),
    ]).reshape(NW * 3, CH)
    rows3 = _head_gather_call(idx3, root2, acc2[0:NP], acc2[NP:])

    loss = _head_call(rows3, fc1_w, fc1_b.reshape(1, D),
                      fc2_w.reshape(D, 1), fc2_b.reshape(1, 1))
    return jnp.reshape(loss, ())


# speedup vs baseline: 13.9371x; 1.0526x over previous
import functools

import jax
import jax.numpy as jnp
from jax import lax
from jax.experimental import pallas as pl
from jax.experimental.pallas import tpu as pltpu
from jax.experimental.pallas import tpu_sc as plsc

N = 10000
NP = 10240
R = 4
P = R * NP
D_IN = 128
D = 64
E = 320000
B = 4096
NC = 2
NS = 16
NW = NC * NS
EPW = 10240
EP = NW * EPW
CH = 128
NCHUNK = EPW // CH
CH2 = 256
NCH2 = EPW // CH2
NCH_A = 48
NCH_B = 2 * NCH2 - NCH_A
EROWS = NW * NCH2
EROWS_PAD = NS * NCH_A + NS * NCH_B + (NCH_A - NCH_B)
PR = P // 128
PRP = 384

_mesh = functools.partial(
    plsc.VectorSubcoreMesh,
    core_axis_name="c", subcore_axis_name="s",
    num_cores=NC, num_subcores=NS,
)
_sc_params = pltpu.CompilerParams(use_tc_tiling_on_sc=False,
                                  needs_layout_passes=False)


def _counts_body(dst_hbm, et_hbm, out_hbm, dst_v, et_v, cv, ibuf, csh):
    cid = lax.axis_index("c")
    sid = lax.axis_index("s")
    wid = sid * NC + cid
    pltpu.sync_copy(dst_hbm.at[pl.ds(wid * NCH2, NCH2)], dst_v)
    pltpu.sync_copy(et_hbm.at[pl.ds(wid * NCH2, NCH2)], et_v)

    z16 = jnp.zeros((16,), jnp.int32)

    def zero_cv(g, _):
        r = g // 8
        c = g - r * 8
        cv[r, pl.ds(c * 16, 16)] = z16
        return 0
    lax.fori_loop(0, PRP * 8, zero_cv, 0)

    def mk_iota(g, _):
        j = g // 8
        l = g - j * 8
        ibuf[j, pl.ds(l * 16, 16)] = lax.iota(jnp.int32, 16) + g * 16
        return 0
    lax.fori_loop(0, (PRP // 128) * 8, mk_iota, 0)

    pltpu.sync_copy(cv.at[pl.ds(sid * (PRP // NS), PRP // NS)],
                    csh.at[pl.ds(sid * (PRP // NS), PRP // NS)])
    plsc.subcore_barrier()

    one = jnp.ones((16,), jnp.int32)

    def count16(g, _):
        j = g // 16
        l = (g - j * 16) * 16
        sl = pl.ds(l, 16)
        kvec = et_v[j, sl] * NP + dst_v[j, sl]
        r = kvec // 128
        plsc.addupdate_scatter(cv, [r, kvec - r * 128], one)
        return 0
    lax.fori_loop(0, EPW // 16, count16, 0)

    def add_chunk(j, _):
        pltpu.sync_copy(cv.at[pl.ds(j * 128, 128)], csh.at[ibuf.at[j]],
                        add=True)
        return 0
    lax.fori_loop(0, PRP // 128, add_chunk, 0)
    plsc.subcore_barrier()

    base = sid * (PR // NS)
    pltpu.sync_copy(csh.at[pl.ds(base, PR // NS)], cv.at[pl.ds(0, PR // NS)])
    pltpu.sync_copy(cv.at[pl.ds(0, PR // NS)],
                    out_hbm.at[pl.ds(cid * PR + base, PR // NS)])


def _counts_call(dst3, et3):
    return pl.kernel(
        _counts_body,
        out_type=jax.ShapeDtypeStruct((NC * PR, 128), jnp.int32),
        mesh=_mesh(),
        compiler_params=_sc_params,
        scratch_types=[
            pltpu.VMEM((NCH2, CH2), jnp.int32),
            pltpu.VMEM((NCH2, CH2), jnp.int32),
            pltpu.VMEM((PRP, 128), jnp.int32),
            pltpu.VMEM((PRP // 128, 128), jnp.int32),
            pltpu.VMEM_SHARED((PRP, 128), jnp.int32),
        ],
    )(dst3, et3)


def _scales_body(src_hbm, dst_hbm, et_hbm, invc_hbm, gk_hbm, s_hbm,
                 gk_v, dst_v, et_v, s_v, invc_v):
    cid = lax.axis_index("c")
    sid = lax.axis_index("s")
    wid = sid * NC + cid
    pltpu.sync_copy(src_hbm.at[pl.ds(wid * NCH2, NCH2)], gk_v)
    pltpu.sync_copy(dst_hbm.at[pl.ds(wid * NCH2, NCH2)], dst_v)
    pltpu.sync_copy(et_hbm.at[pl.ds(wid * NCH2, NCH2)], et_v)
    pltpu.sync_copy(invc_hbm, invc_v)

    def keys_scales(g, _):
        j = g // 16
        l = (g - j * 16) * 16
        sl = pl.ds(l, 16)
        etv = et_v[j, sl] * NP
        sk = etv + dst_v[j, sl]
        gk_v[j, sl] = gk_v[j, sl] + etv
        skr = sk // 128
        s_v[j, sl] = plsc.load_gather(invc_v, [skr, sk - skr * 128])
        return 0
    lax.fori_loop(0, EPW // 16, keys_scales, 0)
    pltpu.sync_copy(gk_v, gk_hbm.at[pl.ds(wid * NCH2, NCH2)])
    pltpu.sync_copy(s_v, s_hbm.at[pl.ds(wid * NCH2, NCH2)])


def _scales_call(src3, dst3, et3, invc):
    return pl.kernel(
        _scales_body,
        out_type=(
            jax.ShapeDtypeStruct((EROWS_PAD, CH2), jnp.int32),
            jax.ShapeDtypeStruct((EROWS_PAD, CH2), jnp.float32),
        ),
        mesh=_mesh(),
        compiler_params=_sc_params,
        scratch_types=[
            pltpu.VMEM((NCH2, CH2), jnp.int32),
            pltpu.VMEM((NCH2, CH2), jnp.int32),
            pltpu.VMEM((NCH2, CH2), jnp.int32),
            pltpu.VMEM((NCH2, CH2), jnp.float32),
            pltpu.VMEM((PR, 128), jnp.float32),
        ],
    )(src3, dst3, et3, invc)


def _edge_body(t_hbm, gk_hbm, dst_hbm, s_hbm, acc_hbm,
               gk_v, dst_v, s_v, bufs, gsems, ssems, acc_sh):
    cid = lax.axis_index("c")
    sid = lax.axis_index("s")
    base = jnp.where(cid == 0, sid * NCH_A, NS * NCH_A + sid * NCH_B)
    nch = jnp.where(cid == 0, NCH_A, NCH_B)
    pltpu.sync_copy(gk_hbm.at[pl.ds(base, NCH_A)], gk_v)
    pltpu.sync_copy(dst_hbm.at[pl.ds(base, NCH_A)], dst_v)
    pltpu.sync_copy(s_hbm.at[pl.ds(base, NCH_A)], s_v)

    zv = jnp.zeros((16,), jnp.float32)
    rows0 = bufs[0]

    def zero_rows(g, _):
        r = g // 4
        c = g - r * 4
        rows0[r, pl.ds(c * 16, 16)] = zv
        return 0
    lax.fori_loop(0, CH * 4, zero_rows, 0)

    def zero_acc(t, _):
        pltpu.sync_copy(rows0.at[pl.ds(0, CH)],
                        acc_sh.at[pl.ds(sid * (NP // NS) + t * CH, CH)])
        return 0
    lax.fori_loop(0, NP // NS // CH, zero_acc, 0)
    plsc.subcore_barrier()

    def scale_chunk(rv, j):
        def scale16(eb, _):
            svec = s_v[j, pl.ds(eb * 16, 16)]
            for i in range(16):
                sval = svec[i]
                e = eb * 16 + i
                for c in range(4):
                    sl = pl.ds(c * 16, 16)
                    rv[e, sl] = rv[e, sl] * sval
            return 0
        lax.fori_loop(0, CH2 // 16, scale16, 0)

    def gidx(j):
        return gk_v.at[j]

    def sidx(j):
        return dst_v.at[j]

    pltpu.async_copy(t_hbm.at[gidx(0)], bufs[0], gsems[0])

    def chunk3(t, _):
        for p in range(3):
            j = t * 3 + p

            @pl.when(j < nch)
            def _():
                nb = (p + 1) % 3

                @pl.when(j >= 2)
                def _():
                    pltpu.make_async_copy(bufs[nb], acc_sh.at[sidx(j - 2)],
                                          ssems[nb]).wait()

                @pl.when(j + 1 < nch)
                def _():
                    pltpu.async_copy(t_hbm.at[gidx(j + 1)], bufs[nb],
                                     gsems[nb])
                pltpu.make_async_copy(t_hbm.at[gidx(j)], bufs[p],
                                      gsems[p]).wait()
                scale_chunk(bufs[p], j)
                pltpu.async_copy(bufs[p], acc_sh.at[sidx(j)], ssems[p],
                                 add=True)
        return 0
    lax.fori_loop(0, (NCH_A + 2) // 3, chunk3, 0)

    @pl.when(cid == 0)
    def _():
        pltpu.make_async_copy(bufs[(NCH_A - 2) % 3], acc_sh.at[sidx(NCH_A - 2)],
                              ssems[(NCH_A - 2) % 3]).wait()
        pltpu.make_async_copy(bufs[(NCH_A - 1) % 3], acc_sh.at[sidx(NCH_A - 1)],
                              ssems[(NCH_A - 1) % 3]).wait()

    @pl.when(cid == 1)
    def _():
        pltpu.make_async_copy(bufs[(NCH_B - 2) % 3], acc_sh.at[sidx(NCH_B - 2)],
                              ssems[(NCH_B - 2) % 3]).wait()
        pltpu.make_async_copy(bufs[(NCH_B - 1) % 3], acc_sh.at[sidx(NCH_B - 1)],
                              ssems[(NCH_B - 1) % 3]).wait()
    plsc.subcore_barrier()

    def export(t, _):
        base = sid * (NP // NS) + t * CH
        pltpu.sync_copy(acc_sh.at[pl.ds(base, CH)], rows0.at[pl.ds(0, CH)])
        pltpu.sync_copy(rows0.at[pl.ds(0, CH)],
                        acc_hbm.at[pl.ds(cid * NP + base, CH)])
        return 0
    lax.fori_loop(0, NP // NS // CH, export, 0)


def _edge_call(t_tab, gk3, dst3, s3):
    return pl.kernel(
        _edge_body,
        out_type=jax.ShapeDtypeStruct((NC * NP, D), jnp.float32),
        mesh=_mesh(),
        compiler_params=_sc_params,
        scratch_types=[
            pltpu.VMEM((NCH_A, CH2), jnp.int32),
            pltpu.VMEM((NCH_A, CH2), jnp.int32),
            pltpu.VMEM((NCH_A, CH2), jnp.float32),
            [pltpu.VMEM((CH2, D), jnp.float32) for _ in range(3)],
            [pltpu.SemaphoreType.DMA for _ in range(3)],
            [pltpu.SemaphoreType.DMA for _ in range(3)],
            pltpu.VMEM_SHARED((NP, D), jnp.float32),
        ],
    )(t_tab, gk3, dst3, s3)


def _head_gather_body(idx_hbm, root_hbm, a0_hbm, a1_hbm, out_hbm,
                      idx_v, r0, r1, r2, sem):
    cid = lax.axis_index("c")
    sid = lax.axis_index("s")
    wid = sid * NC + cid
    pltpu.sync_copy(idx_hbm.at[pl.ds(wid * 3, 3)], idx_v)

    def chunk(j, _):
        pltpu.async_copy(root_hbm.at[idx_v.at[j]], r0, sem).wait()
        pltpu.async_copy(a0_hbm.at[idx_v.at[j]], r1, sem).wait()
        pltpu.async_copy(a1_hbm.at[idx_v.at[j]], r2, sem).wait()

        def addr(r, _):
            for c in range(4):
                sl = pl.ds(c * 16, 16)
                r0[r, sl] = r0[r, sl] + r1[r, sl] + r2[r, sl]
            return 0
        lax.fori_loop(0, CH, addr, 0)
        pltpu.sync_copy(r0, out_hbm.at[pl.ds(wid * 384 + j * CH, CH)])
        return 0
    lax.fori_loop(0, 3, chunk, 0)


def _head_gather_call(idx3, root2, a0, a1):
    return pl.kernel(
        _head_gather_body,
        out_type=jax.ShapeDtypeStruct((3 * B, D), jnp.float32),
        mesh=_mesh(),
        compiler_params=_sc_params,
        scratch_types=[
            pltpu.VMEM((3, CH), jnp.int32),
            pltpu.VMEM((CH, D), jnp.float32),
            pltpu.VMEM((CH, D), jnp.float32),
            pltpu.VMEM((CH, D), jnp.float32),
            pltpu.SemaphoreType.DMA,
        ],
    )(idx3, root2, a0, a1)


def _prep_t_body(x_ref, w_ref, wr_ref, b_ref, t_ref, root_ref):
    x = x_ref[...]
    t_ref[0:N, :] = jnp.dot(x, w_ref[0], preferred_element_type=jnp.float32)
    t_ref[N:NP, :] = jnp.zeros((NP - N, D), jnp.float32)
    root_ref[...] = (
        jnp.dot(x, wr_ref[...], preferred_element_type=jnp.float32)
        + b_ref[...]
    )


def _prep_t_call(x, w1_rel, w1_root, b1r):
    return pl.pallas_call(
        _prep_t_body,
        grid=(R,),
        in_specs=[
            pl.BlockSpec((N, D_IN), lambda r: (0, 0)),
            pl.BlockSpec((1, D_IN, D), lambda r: (r, 0, 0)),
            pl.BlockSpec((D_IN, D), lambda r: (0, 0)),
            pl.BlockSpec((1, D), lambda r: (0, 0)),
        ],
        out_specs=[
            pl.BlockSpec((NP, D), lambda r: (r, 0)),
            pl.BlockSpec((N, D), lambda r: (0, 0)),
        ],
        out_shape=[
            jax.ShapeDtypeStruct((P, D), jnp.float32),
            jax.ShapeDtypeStruct((N, D), jnp.float32),
        ],
    )(x, w1_rel, w1_root, b1r)


def _invc_body(cnt_ref, invc_ref):
    cnt = cnt_ref[...]
    c = (cnt[0:PR] + cnt[PR:2 * PR]).astype(jnp.float32)
    rows = P // 128
    k = (lax.broadcasted_iota(jnp.int32, (rows, 128), 0) * 128
         + lax.broadcasted_iota(jnp.int32, (rows, 128), 1))
    npart = k % NP
    invc_ref[...] = jnp.where(npart < N, 1.0 / jnp.maximum(c, 1.0), 0.0)


def _invc_call(cntp):
    return pl.pallas_call(
        _invc_body,
        out_shape=jax.ShapeDtypeStruct((P // 128, 128), jnp.float32),
    )(cntp)


def _mid_body(root1_ref, acc_ref, w_ref, wr_ref, b_ref, t_ref, root2_ref):
    a = acc_ref[...]
    h = root1_ref[...] + a[0:N, :] + a[NP:NP + N, :]
    h = jnp.maximum(h, 0.0)
    t_ref[0:N, :] = jnp.dot(h, w_ref[0], preferred_element_type=jnp.float32)
    t_ref[N:NP, :] = jnp.zeros((NP - N, D), jnp.float32)
    root2_ref[...] = (
        jnp.dot(h, wr_ref[...], preferred_element_type=jnp.float32)
        + b_ref[...]
    )


def _mid_call(root1, acc1, w2_rel, w2_root, b2r):
    return pl.pallas_call(
        _mid_body,
        grid=(R,),
        in_specs=[
            pl.BlockSpec((N, D), lambda r: (0, 0)),
            pl.BlockSpec((NC * NP, D), lambda r: (0, 0)),
            pl.BlockSpec((1, D, D), lambda r: (r, 0, 0)),
            pl.BlockSpec((D, D), lambda r: (0, 0)),
            pl.BlockSpec((1, D), lambda r: (0, 0)),
        ],
        out_specs=[
            pl.BlockSpec((NP, D), lambda r: (r, 0)),
            pl.BlockSpec((N, D), lambda r: (0, 0)),
        ],
        out_shape=[
            jax.ShapeDtypeStruct((P, D), jnp.float32),
            jax.ShapeDtypeStruct((N, D), jnp.float32),
        ],
    )(root1, acc1, w2_rel, w2_root, b2r)


def _head_body(rows_ref, w1_ref, b1_ref, w2t_ref, b2_ref, out_ref):
    xr = rows_ref[...]
    bill = xr[0:B]
    u1 = xr[B:2 * B]
    u2 = xr[2 * B:3 * B]
    w2b = jnp.broadcast_to(w2t_ref[...], (D, 128))
    b2v = b2_ref[...]

    def mlp(a, b):
        xcat = jnp.concatenate([a, b], axis=1)
        h1 = lax.dot_general(xcat, w1_ref[...], (((1,), (1,)), ((), ())),
                             preferred_element_type=jnp.float32)
        h1 = jnp.maximum(h1 + b1_ref[...], 0.0)
        return jnp.dot(h1, w2b, preferred_element_type=jnp.float32) + b2v[0, 0]

    p = mlp(bill, u1)
    q = mlp(bill, u2)

    def softplus(v):
        return jnp.maximum(v, 0.0) + jnp.log(1.0 + jnp.exp(-jnp.abs(v)))

    v = (softplus(p) - p) + softplus(q)
    s = jnp.sum(v, axis=0, keepdims=True)
    out_ref[...] = s[0:1, 0:1] / (2.0 * B)


def _head_call(rows3, fc1_w, fc1_br, fc2_wt, fc2_br):
    return pl.pallas_call(
        _head_body,
        out_shape=jax.ShapeDtypeStruct((1, 1), jnp.float32),
    )(rows3, fc1_w, fc1_br, fc2_wt, fc2_br)


def kernel(node_embeddings, w1_rel, w1_root, b1, w2_rel, w2_root, b2,
           fc1_w, fc1_b, fc2_w, fc2_b,
           edge_index_combined, edge_type_combined,
           bill_id, user1_id, user2_id):
    src = edge_index_combined[0].astype(jnp.int32)
    dst = edge_index_combined[1].astype(jnp.int32)
    et = edge_type_combined.astype(jnp.int32)
    pad = EROWS_PAD * CH2 - E
    src_p = jnp.concatenate([src, jnp.zeros((pad,), jnp.int32)])
    dst_p = jnp.concatenate([dst, jnp.full((pad,), N, jnp.int32)])
    et_p = jnp.concatenate([et, jnp.zeros((pad,), jnp.int32)])
    src3 = src_p.reshape(EROWS_PAD, CH2)
    dst3 = dst_p.reshape(EROWS_PAD, CH2)
    et3 = et_p.reshape(EROWS_PAD, CH2)

    cntp = _counts_call(dst3, et3)
    t1, root1 = _prep_t_call(
        node_embeddings, w1_rel, w1_root, b1.reshape(1, D))
    invc2 = _invc_call(cntp)

    gk3, s3 = _scales_call(src3, dst3, et3, invc2)
    acc1 = _edge_call(t1, gk3, dst3, s3)
    t2, root2 = _mid_call(root1, acc1, w2_rel, w2_root, b2.reshape(1, D))
    acc2 = _edge_call(t2, gk3, dst3, s3)

    idx3 = jnp.concatenate([
        bill_id.astype(jnp.int32),
        user1_id.astype(jnp.int32),
        user2_id.astype(jnp.int32),
    ]).reshape(NW * 3, CH)
    rows3 = _head_gather_call(idx3, root2, acc2[0:NP], acc2[NP:])

    loss = _head_call(rows3, fc1_w, fc1_b.reshape(1, D),
                      fc2_w.reshape(D, 1), fc2_b.reshape(1, 1))
    return jnp.reshape(loss, ())


# speedup vs baseline: 14.1660x; 1.0164x over previous
import functools

import jax
import jax.numpy as jnp
from jax import lax
from jax.experimental import pallas as pl
from jax.experimental.pallas import tpu as pltpu
from jax.experimental.pallas import tpu_sc as plsc

N = 10000
NP = 10240
R = 4
P = R * NP
D_IN = 128
D = 64
E = 320000
B = 4096
NC = 2
NS = 16
NW = NC * NS
EPW = 10240
EP = NW * EPW
CH = 128
NCHUNK = EPW // CH
CH2 = 256
NCH2 = EPW // CH2
NCH_A = 50
NCH_B = 2 * NCH2 - NCH_A
EROWS = NW * NCH2
EROWS_PAD = NS * NCH_A + NS * NCH_B + (NCH_A - NCH_B)
PR = P // 128
PRP = 384

_mesh = functools.partial(
    plsc.VectorSubcoreMesh,
    core_axis_name="c", subcore_axis_name="s",
    num_cores=NC, num_subcores=NS,
)
_sc_params = pltpu.CompilerParams(use_tc_tiling_on_sc=False,
                                  needs_layout_passes=False)


def _counts_body(dst_hbm, et_hbm, out_hbm, dst_v, et_v, cv, ibuf, csh):
    cid = lax.axis_index("c")
    sid = lax.axis_index("s")
    wid = sid * NC + cid
    pltpu.sync_copy(dst_hbm.at[pl.ds(wid * NCH2, NCH2)], dst_v)
    pltpu.sync_copy(et_hbm.at[pl.ds(wid * NCH2, NCH2)], et_v)

    z16 = jnp.zeros((16,), jnp.int32)

    def zero_cv(g, _):
        r = g // 8
        c = g - r * 8
        cv[r, pl.ds(c * 16, 16)] = z16
        return 0
    lax.fori_loop(0, PRP * 8, zero_cv, 0)

    def mk_iota(g, _):
        j = g // 8
        l = g - j * 8
        ibuf[j, pl.ds(l * 16, 16)] = lax.iota(jnp.int32, 16) + g * 16
        return 0
    lax.fori_loop(0, (PRP // 128) * 8, mk_iota, 0)

    pltpu.sync_copy(cv.at[pl.ds(sid * (PRP // NS), PRP // NS)],
                    csh.at[pl.ds(sid * (PRP // NS), PRP // NS)])
    plsc.subcore_barrier()

    one = jnp.ones((16,), jnp.int32)

    def count16(g, _):
        j = g // 16
        l = (g - j * 16) * 16
        sl = pl.ds(l, 16)
        kvec = et_v[j, sl] * NP + dst_v[j, sl]
        r = kvec // 128
        plsc.addupdate_scatter(cv, [r, kvec - r * 128], one)
        return 0
    lax.fori_loop(0, EPW // 16, count16, 0)

    def add_chunk(j, _):
        pltpu.sync_copy(cv.at[pl.ds(j * 128, 128)], csh.at[ibuf.at[j]],
                        add=True)
        return 0
    lax.fori_loop(0, PRP // 128, add_chunk, 0)
    plsc.subcore_barrier()

    base = sid * (PR // NS)
    pltpu.sync_copy(csh.at[pl.ds(base, PR // NS)], cv.at[pl.ds(0, PR // NS)])
    pltpu.sync_copy(cv.at[pl.ds(0, PR // NS)],
                    out_hbm.at[pl.ds(cid * PR + base, PR // NS)])


def _counts_call(dst3, et3):
    return pl.kernel(
        _counts_body,
        out_type=jax.ShapeDtypeStruct((NC * PR, 128), jnp.int32),
        mesh=_mesh(),
        compiler_params=_sc_params,
        scratch_types=[
            pltpu.VMEM((NCH2, CH2), jnp.int32),
            pltpu.VMEM((NCH2, CH2), jnp.int32),
            pltpu.VMEM((PRP, 128), jnp.int32),
            pltpu.VMEM((PRP // 128, 128), jnp.int32),
            pltpu.VMEM_SHARED((PRP, 128), jnp.int32),
        ],
    )(dst3, et3)


def _scales_body(src_hbm, dst_hbm, et_hbm, invc_hbm, gk_hbm, s_hbm,
                 gk_v, dst_v, et_v, s_v, invc_v):
    cid = lax.axis_index("c")
    sid = lax.axis_index("s")
    wid = sid * NC + cid
    pltpu.sync_copy(src_hbm.at[pl.ds(wid * NCH2, NCH2)], gk_v)
    pltpu.sync_copy(dst_hbm.at[pl.ds(wid * NCH2, NCH2)], dst_v)
    pltpu.sync_copy(et_hbm.at[pl.ds(wid * NCH2, NCH2)], et_v)
    pltpu.sync_copy(invc_hbm, invc_v)

    def keys_scales(g, _):
        j = g // 16
        l = (g - j * 16) * 16
        sl = pl.ds(l, 16)
        etv = et_v[j, sl] * NP
        sk = etv + dst_v[j, sl]
        gk_v[j, sl] = gk_v[j, sl] + etv
        skr = sk // 128
        s_v[j, sl] = plsc.load_gather(invc_v, [skr, sk - skr * 128])
        return 0
    lax.fori_loop(0, EPW // 16, keys_scales, 0)
    pltpu.sync_copy(gk_v, gk_hbm.at[pl.ds(wid * NCH2, NCH2)])
    pltpu.sync_copy(s_v, s_hbm.at[pl.ds(wid * NCH2, NCH2)])


def _scales_call(src3, dst3, et3, invc):
    return pl.kernel(
        _scales_body,
        out_type=(
            jax.ShapeDtypeStruct((EROWS_PAD, CH2), jnp.int32),
            jax.ShapeDtypeStruct((EROWS_PAD, CH2), jnp.float32),
        ),
        mesh=_mesh(),
        compiler_params=_sc_params,
        scratch_types=[
            pltpu.VMEM((NCH2, CH2), jnp.int32),
            pltpu.VMEM((NCH2, CH2), jnp.int32),
            pltpu.VMEM((NCH2, CH2), jnp.int32),
            pltpu.VMEM((NCH2, CH2), jnp.float32),
            pltpu.VMEM((PR, 128), jnp.float32),
        ],
    )(src3, dst3, et3, invc)


def _edge_body(t_hbm, gk_hbm, dst_hbm, s_hbm, acc_hbm,
               gk_v, dst_v, s_v, bufs, gsems, ssems, acc_sh):
    cid = lax.axis_index("c")
    sid = lax.axis_index("s")
    base = jnp.where(cid == 0, sid * NCH_A, NS * NCH_A + sid * NCH_B)
    nch = jnp.where(cid == 0, NCH_A, NCH_B)
    pltpu.sync_copy(gk_hbm.at[pl.ds(base, NCH_A)], gk_v)
    pltpu.sync_copy(dst_hbm.at[pl.ds(base, NCH_A)], dst_v)
    pltpu.sync_copy(s_hbm.at[pl.ds(base, NCH_A)], s_v)

    zv = jnp.zeros((16,), jnp.float32)
    rows0 = bufs[0]

    def zero_rows(g, _):
        r = g // 4
        c = g - r * 4
        rows0[r, pl.ds(c * 16, 16)] = zv
        return 0
    lax.fori_loop(0, CH * 4, zero_rows, 0)

    def zero_acc(t, _):
        pltpu.sync_copy(rows0.at[pl.ds(0, CH)],
                        acc_sh.at[pl.ds(sid * (NP // NS) + t * CH, CH)])
        return 0
    lax.fori_loop(0, NP // NS // CH, zero_acc, 0)
    plsc.subcore_barrier()

    def scale_chunk(rv, j):
        def scale16(eb, _):
            svec = s_v[j, pl.ds(eb * 16, 16)]
            for i in range(16):
                sval = svec[i]
                e = eb * 16 + i
                for c in range(4):
                    sl = pl.ds(c * 16, 16)
                    rv[e, sl] = rv[e, sl] * sval
            return 0
        lax.fori_loop(0, CH2 // 16, scale16, 0)

    def gidx(j):
        return gk_v.at[j]

    def sidx(j):
        return dst_v.at[j]

    pltpu.async_copy(t_hbm.at[gidx(0)], bufs[0], gsems[0])

    def chunk3(t, _):
        for p in range(3):
            j = t * 3 + p

            @pl.when(j < nch)
            def _():
                nb = (p + 1) % 3

                @pl.when(j >= 2)
                def _():
                    pltpu.make_async_copy(bufs[nb], acc_sh.at[sidx(j - 2)],
                                          ssems[nb]).wait()

                @pl.when(j + 1 < nch)
                def _():
                    pltpu.async_copy(t_hbm.at[gidx(j + 1)], bufs[nb],
                                     gsems[nb])
                pltpu.make_async_copy(t_hbm.at[gidx(j)], bufs[p],
                                      gsems[p]).wait()
                scale_chunk(bufs[p], j)
                pltpu.async_copy(bufs[p], acc_sh.at[sidx(j)], ssems[p],
                                 add=True)
        return 0
    lax.fori_loop(0, (NCH_A + 2) // 3, chunk3, 0)

    @pl.when(cid == 0)
    def _():
        pltpu.make_async_copy(bufs[(NCH_A - 2) % 3], acc_sh.at[sidx(NCH_A - 2)],
                              ssems[(NCH_A - 2) % 3]).wait()
        pltpu.make_async_copy(bufs[(NCH_A - 1) % 3], acc_sh.at[sidx(NCH_A - 1)],
                              ssems[(NCH_A - 1) % 3]).wait()

    @pl.when(cid == 1)
    def _():
        pltpu.make_async_copy(bufs[(NCH_B - 2) % 3], acc_sh.at[sidx(NCH_B - 2)],
                              ssems[(NCH_B - 2) % 3]).wait()
        pltpu.make_async_copy(bufs[(NCH_B - 1) % 3], acc_sh.at[sidx(NCH_B - 1)],
                              ssems[(NCH_B - 1) % 3]).wait()
    plsc.subcore_barrier()

    def export(t, _):
        base = sid * (NP // NS) + t * CH
        pltpu.sync_copy(acc_sh.at[pl.ds(base, CH)], rows0.at[pl.ds(0, CH)])
        pltpu.sync_copy(rows0.at[pl.ds(0, CH)],
                        acc_hbm.at[pl.ds(cid * NP + base, CH)])
        return 0
    lax.fori_loop(0, NP // NS // CH, export, 0)


def _edge_call(t_tab, gk3, dst3, s3):
    return pl.kernel(
        _edge_body,
        out_type=jax.ShapeDtypeStruct((NC * NP, D), jnp.float32),
        mesh=_mesh(),
        compiler_params=_sc_params,
        scratch_types=[
            pltpu.VMEM((NCH_A, CH2), jnp.int32),
            pltpu.VMEM((NCH_A, CH2), jnp.int32),
            pltpu.VMEM((NCH_A, CH2), jnp.float32),
            [pltpu.VMEM((CH2, D), jnp.float32) for _ in range(3)],
            [pltpu.SemaphoreType.DMA for _ in range(3)],
            [pltpu.SemaphoreType.DMA for _ in range(3)],
            pltpu.VMEM_SHARED((NP, D), jnp.float32),
        ],
    )(t_tab, gk3, dst3, s3)


def _head_gather_body(idx_hbm, root_hbm, a0_hbm, a1_hbm, out_hbm,
                      idx_v, r0, r1, r2, sem):
    cid = lax.axis_index("c")
    sid = lax.axis_index("s")
    wid = sid * NC + cid
    pltpu.sync_copy(idx_hbm.at[pl.ds(wid * 3, 3)], idx_v)

    def chunk(j, _):
        pltpu.async_copy(root_hbm.at[idx_v.at[j]], r0, sem).wait()
        pltpu.async_copy(a0_hbm.at[idx_v.at[j]], r1, sem).wait()
        pltpu.async_copy(a1_hbm.at[idx_v.at[j]], r2, sem).wait()

        def addr(r, _):
            for c in range(4):
                sl = pl.ds(c * 16, 16)
                r0[r, sl] = r0[r, sl] + r1[r, sl] + r2[r, sl]
            return 0
        lax.fori_loop(0, CH, addr, 0)
        pltpu.sync_copy(r0, out_hbm.at[pl.ds(wid * 384 + j * CH, CH)])
        return 0
    lax.fori_loop(0, 3, chunk, 0)


def _head_gather_call(idx3, root2, a0, a1):
    return pl.kernel(
        _head_gather_body,
        out_type=jax.ShapeDtypeStruct((3 * B, D), jnp.float32),
        mesh=_mesh(),
        compiler_params=_sc_params,
        scratch_types=[
            pltpu.VMEM((3, CH), jnp.int32),
            pltpu.VMEM((CH, D), jnp.float32),
            pltpu.VMEM((CH, D), jnp.float32),
            pltpu.VMEM((CH, D), jnp.float32),
            pltpu.SemaphoreType.DMA,
        ],
    )(idx3, root2, a0, a1)


def _prep_t_body(x_ref, w_ref, wr_ref, b_ref, t_ref, root_ref):
    x = x_ref[...]
    t_ref[0:N, :] = jnp.dot(x, w_ref[0], preferred_element_type=jnp.float32)
    t_ref[N:NP, :] = jnp.zeros((NP - N, D), jnp.float32)
    root_ref[...] = (
        jnp.dot(x, wr_ref[...], preferred_element_type=jnp.float32)
        + b_ref[...]
    )


def _prep_t_call(x, w1_rel, w1_root, b1r):
    return pl.pallas_call(
        _prep_t_body,
        grid=(R,),
        in_specs=[
            pl.BlockSpec((N, D_IN), lambda r: (0, 0)),
            pl.BlockSpec((1, D_IN, D), lambda r: (r, 0, 0)),
            pl.BlockSpec((D_IN, D), lambda r: (0, 0)),
            pl.BlockSpec((1, D), lambda r: (0, 0)),
        ],
        out_specs=[
            pl.BlockSpec((NP, D), lambda r: (r, 0)),
            pl.BlockSpec((N, D), lambda r: (0, 0)),
        ],
        out_shape=[
            jax.ShapeDtypeStruct((P, D), jnp.float32),
            jax.ShapeDtypeStruct((N, D), jnp.float32),
        ],
    )(x, w1_rel, w1_root, b1r)


def _invc_body(cnt_ref, invc_ref):
    cnt = cnt_ref[...]
    c = (cnt[0:PR] + cnt[PR:2 * PR]).astype(jnp.float32)
    rows = P // 128
    k = (lax.broadcasted_iota(jnp.int32, (rows, 128), 0) * 128
         + lax.broadcasted_iota(jnp.int32, (rows, 128), 1))
    npart = k % NP
    invc_ref[...] = jnp.where(npart < N, 1.0 / jnp.maximum(c, 1.0), 0.0)


def _invc_call(cntp):
    return pl.pallas_call(
        _invc_body,
        out_shape=jax.ShapeDtypeStruct((P // 128, 128), jnp.float32),
    )(cntp)


def _mid_body(root1_ref, acc_ref, w_ref, wr_ref, b_ref, t_ref, root2_ref):
    a = acc_ref[...]
    h = root1_ref[...] + a[0:N, :] + a[NP:NP + N, :]
    h = jnp.maximum(h, 0.0)
    t_ref[0:N, :] = jnp.dot(h, w_ref[0], preferred_element_type=jnp.float32)
    t_ref[N:NP, :] = jnp.zeros((NP - N, D), jnp.float32)
    root2_ref[...] = (
        jnp.dot(h, wr_ref[...], preferred_element_type=jnp.float32)
        + b_ref[...]
    )


def _mid_call(root1, acc1, w2_rel, w2_root, b2r):
    return pl.pallas_call(
        _mid_body,
        grid=(R,),
        in_specs=[
            pl.BlockSpec((N, D), lambda r: (0, 0)),
            pl.BlockSpec((NC * NP, D), lambda r: (0, 0)),
            pl.BlockSpec((1, D, D), lambda r: (r, 0, 0)),
            pl.BlockSpec((D, D), lambda r: (0, 0)),
            pl.BlockSpec((1, D), lambda r: (0, 0)),
        ],
        out_specs=[
            pl.BlockSpec((NP, D), lambda r: (r, 0)),
            pl.BlockSpec((N, D), lambda r: (0, 0)),
        ],
        out_shape=[
            jax.ShapeDtypeStruct((P, D), jnp.float32),
            jax.ShapeDtypeStruct((N, D), jnp.float32),
        ],
    )(root1, acc1, w2_rel, w2_root, b2r)


def _head_body(rows_ref, w1_ref, b1_ref, w2t_ref, b2_ref, out_ref):
    xr = rows_ref[...]
    bill = xr[0:B]
    u1 = xr[B:2 * B]
    u2 = xr[2 * B:3 * B]
    w2b = jnp.broadcast_to(w2t_ref[...], (D, 128))
    b2v = b2_ref[...]

    def mlp(a, b):
        xcat = jnp.concatenate([a, b], axis=1)
        h1 = lax.dot_general(xcat, w1_ref[...], (((1,), (1,)), ((), ())),
                             preferred_element_type=jnp.float32)
        h1 = jnp.maximum(h1 + b1_ref[...], 0.0)
        return jnp.dot(h1, w2b, preferred_element_type=jnp.float32) + b2v[0, 0]

    p = mlp(bill, u1)
    q = mlp(bill, u2)

    def softplus(v):
        return jnp.maximum(v, 0.0) + jnp.log(1.0 + jnp.exp(-jnp.abs(v)))

    v = (softplus(p) - p) + softplus(q)
    s = jnp.sum(v, axis=0, keepdims=True)
    out_ref[...] = s[0:1, 0:1] / (2.0 * B)


def _head_call(rows3, fc1_w, fc1_br, fc2_wt, fc2_br):
    return pl.pallas_call(
        _head_body,
        out_shape=jax.ShapeDtypeStruct((1, 1), jnp.float32),
    )(rows3, fc1_w, fc1_br, fc2_wt, fc2_br)


def kernel(node_embeddings, w1_rel, w1_root, b1, w2_rel, w2_root, b2,
           fc1_w, fc1_b, fc2_w, fc2_b,
           edge_index_combined, edge_type_combined,
           bill_id, user1_id, user2_id):
    src = edge_index_combined[0].astype(jnp.int32)
    dst = edge_index_combined[1].astype(jnp.int32)
    et = edge_type_combined.astype(jnp.int32)
    pad = EROWS_PAD * CH2 - E
    src_p = jnp.concatenate([src, jnp.zeros((pad,), jnp.int32)])
    dst_p = jnp.concatenate([dst, jnp.full((pad,), N, jnp.int32)])
    et_p = jnp.concatenate([et, jnp.zeros((pad,), jnp.int32)])
    src3 = src_p.reshape(EROWS_PAD, CH2)
    dst3 = dst_p.reshape(EROWS_PAD, CH2)
    et3 = et_p.reshape(EROWS_PAD, CH2)

    cntp = _counts_call(dst3, et3)
    t1, root1 = _prep_t_call(
        node_embeddings, w1_rel, w1_root, b1.reshape(1, D))
    invc2 = _invc_call(cntp)

    gk3, s3 = _scales_call(src3, dst3, et3, invc2)
    acc1 = _edge_call(t1, gk3, dst3, s3)
    t2, root2 = _mid_call(root1, acc1, w2_rel, w2_root, b2.reshape(1, D))
    acc2 = _edge_call(t2, gk3, dst3, s3)

    idx3 = jnp.concatenate([
        bill_id.astype(jnp.int32),
        user1_id.astype(jnp.int32),
        user2_id.astype(jnp.int32),
    ]).reshape(NW * 3, CH)
    rows3 = _head_gather_call(idx3, root2, acc2[0:NP], acc2[NP:])

    loss = _head_call(rows3, fc1_w, fc1_b.reshape(1, D),
                      fc2_w.reshape(D, 1), fc2_b.reshape(1, 1))
    return jnp.reshape(loss, ())
